# Initial kernel scaffold; baseline (speedup 1.0000x reference)
import jax
import jax.numpy as jnp
from jax.experimental import pallas as pl


def kernel(x_edges, x_edges_values, x_nodes_coord, x_tour, x_best_tour, x_tour_directed, params):
    raise NotImplementedError("write your pallas kernel here")


# speedup vs baseline: 1.8801x; 1.8801x over previous
import functools

import numpy as np
import jax
import jax.numpy as jnp
from jax import lax
from jax.experimental import pallas as pl
from jax.experimental.pallas import tpu as pltpu
from jax.experimental.pallas import tpu_sc as plsc

B, V, H = 8, 100, 128
HH = H // 2
P = V * (V - 1) // 2
PP = 4992
NTAB = 4
R_ROWS = NTAB * B * PP
NW = 32
CHUNK = 128
ROWS_PER_W = R_ROWS // NW
NCHUNK = ROWS_PER_W // CHUNK

_RS, _CS = np.triu_indices(V, 1)
RS_PAD = np.concatenate([_RS, np.zeros(PP - P, np.int64)]).astype(np.int32)
CS_PAD = np.concatenate([_CS, np.zeros(PP - P, np.int64)]).astype(np.int32)
TRIU = np.triu(np.ones((V, V), bool), 1)


def _einit_body(vals_ref, tour_ref, best_ref, wev_ref, emb0_ref, emb1_ref,
                e0_ref):
    vals = vals_ref[0]
    ev = vals * wev_ref[0][None, :]
    t = tour_ref[0]
    bt = best_ref[0]
    tags = (jnp.where(t > 0, emb0_ref[1:2, :], emb0_ref[0:1, :])
            + jnp.where(bt > 0, emb1_ref[1:2, :], emb1_ref[0:1, :]))
    e0_ref[0] = jnp.concatenate([ev, tags], axis=-1)


def _xinit_body(coord_ref, wn_ref, x0_ref):
    cr = coord_ref[0]
    x0_ref[0] = cr[:, 0:1] * wn_ref[0:1, :] + cr[:, 1:2] * wn_ref[1:2, :]


def _node_body(x_ref, vew_ref, veb_ref, unw_ref, unb_ref, vnw_ref, vnb_ref,
               vx_ref, ux_ref, vn_ref):
    x = x_ref[0]
    vx_ref[0] = jnp.dot(x, vew_ref[...], preferred_element_type=jnp.float32) + veb_ref[...]
    ux_ref[0] = jnp.dot(x, unw_ref[...], preferred_element_type=jnp.float32) + unb_ref[...]
    vn_ref[0] = jnp.dot(x, vnw_ref[...], preferred_element_type=jnp.float32) + vnb_ref[...]


def _edge_body(e_ref, vxf_ref, vxr_ref, uxr_ref, vnf_ref, xr_ref,
               uew_ref, ueb_ref, ge_ref, be_ref, gn_ref, bn_ref,
               eo_ref, xo_ref):
    e_row = e_ref[0, 0]
    ue = jnp.dot(e_row, uew_ref[...], preferred_element_type=jnp.float32) + ueb_ref[...]
    e_tmp = ue + vxr_ref[0, 0] + vxf_ref[0]
    gate = 1.0 / (1.0 + jnp.exp(-e_tmp))
    num = jnp.sum(gate * vnf_ref[0], axis=0, keepdims=True)
    den = 1e-20 + jnp.sum(gate, axis=0, keepdims=True)
    x_tmp = uxr_ref[0, 0] + num / den
    eo_ref[0, 0] = e_row + jnp.maximum(e_tmp * ge_ref[...] + be_ref[...], 0.0)
    xo_ref[0, 0] = xr_ref[0, 0] + jnp.maximum(x_tmp * gn_ref[...] + bn_ref[...], 0.0)


def _sc_gather_body(table_hbm, idx_hbm, out_hbm, idx_v, rows_v, sem):
    wid = lax.axis_index("s") * 2 + lax.axis_index("c")
    base = wid * ROWS_PER_W

    def chunk(i, carry):
        off = pl.multiple_of(base + i * CHUNK, CHUNK)
        pltpu.sync_copy(idx_hbm.at[pl.ds(off, CHUNK)], idx_v)
        pltpu.async_copy(table_hbm.at[idx_v], rows_v, sem).wait()
        pltpu.sync_copy(rows_v, out_hbm.at[pl.ds(off, CHUNK)])
        return carry

    lax.fori_loop(0, NCHUNK, chunk, 0)


_sc_gather = functools.partial(
    pl.kernel,
    out_type=jax.ShapeDtypeStruct((R_ROWS, H), jnp.float32),
    mesh=plsc.VectorSubcoreMesh(core_axis_name="c", subcore_axis_name="s"),
    scratch_types=[
        pltpu.VMEM((CHUNK,), jnp.int32),
        pltpu.VMEM((CHUNK, H), jnp.float32),
        pltpu.SemaphoreType.DMA,
    ],
)(_sc_gather_body)


def _mlp_body(o1_ref, o2_ref, g1_ref, g2_ref, noise_ref,
              wa_ref, wb_ref, wc_ref, wd_ref, bp_ref,
              w1_ref, b1_ref, w2_ref, b2_ref, w3_ref, b3_ref,
              wo_ref, bo_ref,
              act_ref, pi_ref):
    f32 = jnp.float32
    h = (jnp.dot(o1_ref[0, 0], wa_ref[...], preferred_element_type=f32)
         + jnp.dot(o2_ref[0, 0], wb_ref[...], preferred_element_type=f32)
         + jnp.dot(g1_ref[0, 0], wc_ref[...], preferred_element_type=f32)
         + jnp.dot(g2_ref[0, 0], wd_ref[...], preferred_element_type=f32)
         + bp_ref[...])
    h = jnp.maximum(jnp.dot(h, w1_ref[...], preferred_element_type=f32) + b1_ref[...], 0.0)
    h = jnp.maximum(jnp.dot(h, w2_ref[...], preferred_element_type=f32) + b2_ref[...], 0.0)
    h = jnp.maximum(jnp.dot(h, w3_ref[...], preferred_element_type=f32) + b3_ref[...], 0.0)
    logits = jnp.dot(h, wo_ref[...], preferred_element_type=f32) + bo_ref[...]
    rowid = lax.broadcasted_iota(jnp.int32, (PP, 1), 0)
    logits = jnp.where(rowid < P, logits, f32(-1e30))
    z = logits + noise_ref[0]
    maxz = jnp.max(z)
    action = jnp.min(jnp.where(z >= maxz, rowid, jnp.int32(PP)))
    m = jnp.max(logits)
    lse = m + jnp.log(jnp.sum(jnp.exp(logits - m)))
    logit_a = jnp.sum(jnp.where(rowid == action, logits, 0.0))
    act_ref[0] = action[None, None]
    pi_ref[0] = (logit_a - lse)[None, None]


def _full(shape):
    nd = len(shape)
    return pl.BlockSpec(shape, lambda *a: (0,) * nd)


def kernel(x_edges, x_edges_values, x_nodes_coord, x_tour, x_best_tour,
           x_tour_directed, params):
    p = params
    f32 = jnp.float32
    cbn = np.float32(1.0 / np.sqrt(1.0 + 1e-5))
    xt = x_tour.astype(jnp.int32)
    xb = x_best_tour.astype(jnp.int32)

    vals4 = x_edges_values.reshape(B, V, V, 1)
    t4 = xt.reshape(B, V, V, 1)
    b4 = xb.reshape(B, V, V, 1)
    wev = p['W_evals'].reshape(1, HH)
    e0 = pl.pallas_call(
        _einit_body,
        grid=(B,),
        in_specs=[
            pl.BlockSpec((1, V, V, 1), lambda b: (b, 0, 0, 0)),
            pl.BlockSpec((1, V, V, 1), lambda b: (b, 0, 0, 0)),
            pl.BlockSpec((1, V, V, 1), lambda b: (b, 0, 0, 0)),
            _full((1, HH)), _full((3, HH)), _full((3, HH)),
        ],
        out_specs=pl.BlockSpec((1, V, V, H), lambda b: (b, 0, 0, 0)),
        out_shape=jax.ShapeDtypeStruct((B, V, V, H), f32),
    )(vals4, t4, b4, wev, p['emb0'], p['emb1'])

    x = pl.pallas_call(
        _xinit_body,
        grid=(B,),
        in_specs=[pl.BlockSpec((1, V, 2), lambda b: (b, 0, 0)),
                  _full((2, H))],
        out_specs=pl.BlockSpec((1, V, H), lambda b: (b, 0, 0)),
        out_shape=jax.ShapeDtypeStruct((B, V, H), f32),
    )(x_nodes_coord, p['W_nodes'])

    e = e0
    for lp in p['layers']:
        vx, ux, vn = pl.pallas_call(
            _node_body,
            grid=(B,),
            in_specs=[pl.BlockSpec((1, V, H), lambda b: (b, 0, 0)),
                      _full((H, H)), _full((1, H)),
                      _full((H, H)), _full((1, H)),
                      _full((H, H)), _full((1, H))],
            out_specs=[pl.BlockSpec((1, V, H), lambda b: (b, 0, 0))] * 3,
            out_shape=[jax.ShapeDtypeStruct((B, V, H), f32)] * 3,
        )(x, lp['Ve'][0], lp['Ve'][1].reshape(1, H),
          lp['Un'][0], lp['Un'][1].reshape(1, H),
          lp['Vn'][0], lp['Vn'][1].reshape(1, H))

        ge = (lp['bn_e'][0] * cbn).reshape(1, H)
        be = lp['bn_e'][1].reshape(1, H)
        gn = (lp['bn_n'][0] * cbn).reshape(1, H)
        bn = lp['bn_n'][1].reshape(1, H)
        e, x4 = pl.pallas_call(
            _edge_body,
            grid=(B, V),
            in_specs=[
                pl.BlockSpec((1, 1, V, H), lambda b, i: (b, i, 0, 0)),
                pl.BlockSpec((1, V, H), lambda b, i: (b, 0, 0)),
                pl.BlockSpec((1, 1, 1, H), lambda b, i: (b, i, 0, 0)),
                pl.BlockSpec((1, 1, 1, H), lambda b, i: (b, i, 0, 0)),
                pl.BlockSpec((1, V, H), lambda b, i: (b, 0, 0)),
                pl.BlockSpec((1, 1, 1, H), lambda b, i: (b, i, 0, 0)),
                pl.BlockSpec((H, H), lambda b, i: (0, 0)),
                pl.BlockSpec((1, H), lambda b, i: (0, 0)),
                pl.BlockSpec((1, H), lambda b, i: (0, 0)),
                pl.BlockSpec((1, H), lambda b, i: (0, 0)),
                pl.BlockSpec((1, H), lambda b, i: (0, 0)),
                pl.BlockSpec((1, H), lambda b, i: (0, 0)),
            ],
            out_specs=[pl.BlockSpec((1, 1, V, H), lambda b, i: (b, i, 0, 0)),
                       pl.BlockSpec((1, 1, 1, H), lambda b, i: (b, i, 0, 0))],
            out_shape=[jax.ShapeDtypeStruct((B, V, V, H), f32),
                       jax.ShapeDtypeStruct((B, V, 1, H), f32)],
        )(e, vx, vx.reshape(B, V, 1, H), ux.reshape(B, V, 1, H), vn,
          x.reshape(B, V, 1, H),
          lp['Ue'][0], lp['Ue'][1].reshape(1, H), ge, be, gn, bn)
        x = x4.reshape(B, V, H)

    mask = (xt > 0) & jnp.asarray(TRIU)
    flat = mask.reshape(B, V * V)
    pos = jnp.where(flat, jnp.arange(V * V, dtype=jnp.int32), jnp.int32(V * V))
    spos = jnp.sort(pos, axis=1)[:, :V]
    i_e = spos // V
    j_e = spos % V
    d = jnp.take_along_axis(x_tour_directed.reshape(B, V * V), spos, axis=1)
    U = jnp.where(d, i_e, j_e)
    Vv = jnp.where(d, j_e, i_e)

    boff = (jnp.arange(B, dtype=jnp.int32) * (V * V))[:, None]
    Uk1, Uk2 = U[:, RS_PAD], U[:, CS_PAD]
    Vk1, Vk2 = Vv[:, RS_PAD], Vv[:, CS_PAD]
    idx_all = jnp.stack([
        boff + Uk1 * V + Vk1,
        boff + Uk2 * V + Vk2,
        boff + Uk1 * V + Uk2,
        boff + Vk1 * V + Vk2,
    ]).reshape(R_ROWS)

    table = e.reshape(B * V * V, H)
    rows = _sc_gather(table, idx_all)
    quad = rows.reshape(NTAB, B, PP, H)

    noise = jax.random.gumbel(jax.random.key(42), (B, P), f32)
    noise = jnp.pad(noise, ((0, 0), (0, PP - P))).reshape(B, PP, 1)
    Wp, bp = p['pre_act']
    w1, b1 = p['act_hidden'][0]
    w2, b2 = p['act_hidden'][1]
    w3, b3 = p['act_hidden'][2]
    wo, bo = p['act_out']
    tab_spec = lambda t: pl.BlockSpec((1, 1, PP, H), lambda b, _t=t: (_t, b, 0, 0))
    act2, pi2 = pl.pallas_call(
        _mlp_body,
        grid=(B,),
        in_specs=[
            tab_spec(0), tab_spec(1), tab_spec(2), tab_spec(3),
            pl.BlockSpec((1, PP, 1), lambda b: (b, 0, 0)),
            _full((H, H)), _full((H, H)), _full((H, H)), _full((H, H)),
            _full((1, H)),
            _full((H, H)), _full((1, H)),
            _full((H, H)), _full((1, H)),
            _full((H, H)), _full((1, H)),
            _full((H, 1)), _full((1, 1)),
        ],
        out_specs=[pl.BlockSpec((1, 1, 1), lambda b: (b, 0, 0)),
                   pl.BlockSpec((1, 1, 1), lambda b: (b, 0, 0))],
        out_shape=[jax.ShapeDtypeStruct((B, 1, 1), jnp.int32),
                   jax.ShapeDtypeStruct((B, 1, 1), f32)],
    )(quad, quad, quad, quad, noise,
      Wp[0:H], Wp[H:2 * H], Wp[2 * H:3 * H], Wp[3 * H:4 * H], bp.reshape(1, H),
      w1, b1.reshape(1, H), w2, b2.reshape(1, H), w3, b3.reshape(1, H),
      wo, bo.reshape(1, 1))

    actions = act2[:, 0, 0]
    pi = pi2[:, 0, 0]

    k1 = jnp.asarray(RS_PAD)[actions]
    k2 = jnp.asarray(CS_PAD)[actions]
    barange = jnp.arange(B, dtype=jnp.int32)

    def edge_row(kk):
        return jnp.stack([
            barange,
            jnp.take_along_axis(i_e, kk[:, None], axis=1)[:, 0],
            jnp.take_along_axis(j_e, kk[:, None], axis=1)[:, 0],
        ], axis=1)

    edges = jnp.stack([edge_row(k1), edge_row(k2)], axis=1)
    return edges, pi, actions


# speedup vs baseline: 4.4988x; 2.3928x over previous
import functools

import numpy as np
import jax
import jax.numpy as jnp
from jax import lax
from jax.experimental import pallas as pl
from jax.experimental.pallas import tpu as pltpu
from jax.experimental.pallas import tpu_sc as plsc

B, V, H = 8, 100, 128
HH = H // 2
P = V * (V - 1) // 2
PP = 4992
NTAB = 4
R_ROWS = NTAB * B * PP
NW = 32
CHUNK = 128
ROWS_PER_W = R_ROWS // NW
NCHUNK = ROWS_PER_W // CHUNK
RT = 10

_RS, _CS = np.triu_indices(V, 1)
RS_PAD = np.concatenate([_RS, np.zeros(PP - P, np.int64)]).astype(np.int32)
CS_PAD = np.concatenate([_CS, np.zeros(PP - P, np.int64)]).astype(np.int32)


def _node1_body(coord_ref, wn_ref, vew_ref, veb_ref, unw_ref, unb_ref,
                vnw_ref, vnb_ref, x0_ref, vx_ref, ux_ref, vn_ref):
    cr = coord_ref[0]
    x = cr[:, 0:1] * wn_ref[0:1, :] + cr[:, 1:2] * wn_ref[1:2, :]
    x0_ref[0] = x
    vx_ref[0] = jnp.dot(x, vew_ref[...], preferred_element_type=jnp.float32) + veb_ref[...]
    ux_ref[0] = jnp.dot(x, unw_ref[...], preferred_element_type=jnp.float32) + unb_ref[...]
    vn_ref[0] = jnp.dot(x, vnw_ref[...], preferred_element_type=jnp.float32) + vnb_ref[...]


def _node_body(x_ref, vew_ref, veb_ref, unw_ref, unb_ref, vnw_ref, vnb_ref,
               vx_ref, ux_ref, vn_ref):
    x = x_ref[0]
    vx_ref[0] = jnp.dot(x, vew_ref[...], preferred_element_type=jnp.float32) + veb_ref[...]
    ux_ref[0] = jnp.dot(x, unw_ref[...], preferred_element_type=jnp.float32) + unb_ref[...]
    vn_ref[0] = jnp.dot(x, vnw_ref[...], preferred_element_type=jnp.float32) + vnb_ref[...]


def _edge_update(e_row, r, vxf, vxr_ref, uxr_ref, vnf, xr_ref,
                 uew_ref, ueb_ref, ge_ref, be_ref, gn_ref, bn_ref,
                 eo_ref, xo_ref):
    ue = jnp.dot(e_row, uew_ref[...], preferred_element_type=jnp.float32) + ueb_ref[...]
    e_tmp = ue + vxr_ref[0, r] + vxf
    gate = 1.0 / (1.0 + jnp.exp(-e_tmp))
    num = jnp.sum(gate * vnf, axis=0, keepdims=True)
    den = 1e-20 + jnp.sum(gate, axis=0, keepdims=True)
    x_tmp = uxr_ref[0, r] + num / den
    eo_ref[0, r] = e_row + jnp.maximum(e_tmp * ge_ref[...] + be_ref[...], 0.0)
    xo_ref[0, r] = xr_ref[0, r] + jnp.maximum(x_tmp * gn_ref[...] + bn_ref[...], 0.0)


def _edge1_body(vals_ref, tour_ref, best_ref, wev_ref, emb0_ref, emb1_ref,
                vxf_ref, vxr_ref, uxr_ref, vnf_ref, xr_ref,
                uew_ref, ueb_ref, ge_ref, be_ref, gn_ref, bn_ref,
                eo_ref, xo_ref):
    vxf = vxf_ref[0]
    vnf = vnf_ref[0]
    for r in range(RT):
        ev = vals_ref[0, r] * wev_ref[...]
        tags = (jnp.where(tour_ref[0, r] > 0, emb0_ref[1:2, :], emb0_ref[0:1, :])
                + jnp.where(best_ref[0, r] > 0, emb1_ref[1:2, :], emb1_ref[0:1, :]))
        e_row = jnp.concatenate([ev, tags], axis=-1)
        _edge_update(e_row, r, vxf, vxr_ref, uxr_ref, vnf, xr_ref,
                     uew_ref, ueb_ref, ge_ref, be_ref, gn_ref, bn_ref,
                     eo_ref, xo_ref)


def _edge_body(e_ref, vxf_ref, vxr_ref, uxr_ref, vnf_ref, xr_ref,
               uew_ref, ueb_ref, ge_ref, be_ref, gn_ref, bn_ref,
               eo_ref, xo_ref):
    vxf = vxf_ref[0]
    vnf = vnf_ref[0]
    for r in range(RT):
        _edge_update(e_ref[0, r], r, vxf, vxr_ref, uxr_ref, vnf, xr_ref,
                     uew_ref, ueb_ref, ge_ref, be_ref, gn_ref, bn_ref,
                     eo_ref, xo_ref)


def _sc_gather_body(table_hbm, idx_hbm, out_hbm, idx_v, rows_v, sem):
    wid = lax.axis_index("s") * 2 + lax.axis_index("c")
    base = wid * ROWS_PER_W

    def chunk(i, carry):
        off = pl.multiple_of(base + i * CHUNK, CHUNK)
        pltpu.sync_copy(idx_hbm.at[pl.ds(off, CHUNK)], idx_v)
        pltpu.async_copy(table_hbm.at[idx_v], rows_v, sem).wait()
        pltpu.sync_copy(rows_v, out_hbm.at[pl.ds(off, CHUNK)])
        return carry

    lax.fori_loop(0, NCHUNK, chunk, 0)


_sc_gather = functools.partial(
    pl.kernel,
    out_type=jax.ShapeDtypeStruct((R_ROWS, H), jnp.float32),
    mesh=plsc.VectorSubcoreMesh(core_axis_name="c", subcore_axis_name="s"),
    scratch_types=[
        pltpu.VMEM((CHUNK,), jnp.int32),
        pltpu.VMEM((CHUNK, H), jnp.float32),
        pltpu.SemaphoreType.DMA,
    ],
)(_sc_gather_body)


def _mlp_body(o1_ref, o2_ref, g1_ref, g2_ref, noise_ref,
              wa_ref, wb_ref, wc_ref, wd_ref, bp_ref,
              w1_ref, b1_ref, w2_ref, b2_ref, w3_ref, b3_ref,
              wo_ref, bo_ref,
              act_ref, pi_ref):
    f32 = jnp.float32
    h = (jnp.dot(o1_ref[0, 0], wa_ref[...], preferred_element_type=f32)
         + jnp.dot(o2_ref[0, 0], wb_ref[...], preferred_element_type=f32)
         + jnp.dot(g1_ref[0, 0], wc_ref[...], preferred_element_type=f32)
         + jnp.dot(g2_ref[0, 0], wd_ref[...], preferred_element_type=f32)
         + bp_ref[...])
    h = jnp.maximum(jnp.dot(h, w1_ref[...], preferred_element_type=f32) + b1_ref[...], 0.0)
    h = jnp.maximum(jnp.dot(h, w2_ref[...], preferred_element_type=f32) + b2_ref[...], 0.0)
    h = jnp.maximum(jnp.dot(h, w3_ref[...], preferred_element_type=f32) + b3_ref[...], 0.0)
    logits = jnp.dot(h, wo_ref[...], preferred_element_type=f32) + bo_ref[...]
    rowid = lax.broadcasted_iota(jnp.int32, (PP, 1), 0)
    logits = jnp.where(rowid < P, logits, f32(-1e30))
    z = logits + noise_ref[0]
    maxz = jnp.max(z)
    action = jnp.min(jnp.where(z >= maxz, rowid, jnp.int32(PP)))
    m = jnp.max(logits)
    lse = m + jnp.log(jnp.sum(jnp.exp(logits - m)))
    logit_a = jnp.sum(jnp.where(rowid == action, logits, 0.0))
    act_ref[0] = action[None, None]
    pi_ref[0] = (logit_a - lse)[None, None]


def _full(shape):
    nd = len(shape)
    return pl.BlockSpec(shape, lambda *a: (0,) * nd)


def kernel(x_edges, x_edges_values, x_nodes_coord, x_tour, x_best_tour,
           x_tour_directed, params):
    p = params
    f32 = jnp.float32
    cbn = np.float32(1.0 / np.sqrt(1.0 + 1e-5))
    xt = x_tour.astype(jnp.int32)
    xb = x_best_tour.astype(jnp.int32)

    vals4 = x_edges_values.reshape(B, V, V, 1)
    t4 = xt.reshape(B, V, V, 1)
    b4 = xb.reshape(B, V, V, 1)
    wev = p['W_evals'].reshape(1, HH)

    node_w_specs = [_full((H, H)), _full((1, H)),
                    _full((H, H)), _full((1, H)),
                    _full((H, H)), _full((1, H))]
    bvh_spec = pl.BlockSpec((1, V, H), lambda b: (b, 0, 0))

    e = None
    x = None
    for li, lp in enumerate(p['layers']):
        node_w = (lp['Ve'][0], lp['Ve'][1].reshape(1, H),
                  lp['Un'][0], lp['Un'][1].reshape(1, H),
                  lp['Vn'][0], lp['Vn'][1].reshape(1, H))
        if li == 0:
            x, vx, ux, vn = pl.pallas_call(
                _node1_body,
                grid=(B,),
                in_specs=[pl.BlockSpec((1, V, 2), lambda b: (b, 0, 0)),
                          _full((2, H))] + node_w_specs,
                out_specs=[bvh_spec] * 4,
                out_shape=[jax.ShapeDtypeStruct((B, V, H), f32)] * 4,
            )(x_nodes_coord, p['W_nodes'], *node_w)
        else:
            vx, ux, vn = pl.pallas_call(
                _node_body,
                grid=(B,),
                in_specs=[bvh_spec] + node_w_specs,
                out_specs=[bvh_spec] * 3,
                out_shape=[jax.ShapeDtypeStruct((B, V, H), f32)] * 3,
            )(x, *node_w)

        ge = (lp['bn_e'][0] * cbn).reshape(1, H)
        be = lp['bn_e'][1].reshape(1, H)
        gn = (lp['bn_n'][0] * cbn).reshape(1, H)
        bn = lp['bn_n'][1].reshape(1, H)
        row_spec = pl.BlockSpec((1, RT, 1, H), lambda b, i: (b, i, 0, 0))
        shared_specs = [
            pl.BlockSpec((1, V, H), lambda b, i: (b, 0, 0)),
            row_spec,
            row_spec,
            pl.BlockSpec((1, V, H), lambda b, i: (b, 0, 0)),
            row_spec,
            pl.BlockSpec((H, H), lambda b, i: (0, 0)),
            pl.BlockSpec((1, H), lambda b, i: (0, 0)),
            pl.BlockSpec((1, H), lambda b, i: (0, 0)),
            pl.BlockSpec((1, H), lambda b, i: (0, 0)),
            pl.BlockSpec((1, H), lambda b, i: (0, 0)),
            pl.BlockSpec((1, H), lambda b, i: (0, 0)),
        ]
        shared_args = (vx, vx.reshape(B, V, 1, H), ux.reshape(B, V, 1, H),
                       vn, x.reshape(B, V, 1, H),
                       lp['Ue'][0], lp['Ue'][1].reshape(1, H), ge, be, gn, bn)
        out_specs = [pl.BlockSpec((1, RT, V, H), lambda b, i: (b, i, 0, 0)),
                     row_spec]
        out_shape = [jax.ShapeDtypeStruct((B, V, V, H), f32),
                     jax.ShapeDtypeStruct((B, V, 1, H), f32)]
        if li == 0:
            e, x4 = pl.pallas_call(
                _edge1_body,
                grid=(B, V // RT),
                in_specs=[
                    pl.BlockSpec((1, RT, V, 1), lambda b, i: (b, i, 0, 0)),
                    pl.BlockSpec((1, RT, V, 1), lambda b, i: (b, i, 0, 0)),
                    pl.BlockSpec((1, RT, V, 1), lambda b, i: (b, i, 0, 0)),
                    _full((1, HH)), _full((3, HH)), _full((3, HH)),
                ] + shared_specs,
                out_specs=out_specs,
                out_shape=out_shape,
            )(vals4, t4, b4, wev, p['emb0'], p['emb1'], *shared_args)
        else:
            e, x4 = pl.pallas_call(
                _edge_body,
                grid=(B, V // RT),
                in_specs=[pl.BlockSpec((1, RT, V, H), lambda b, i: (b, i, 0, 0))]
                + shared_specs,
                out_specs=out_specs,
                out_shape=out_shape,
            )(e, *shared_args)
        x = x4.reshape(B, V, H)

    first = jnp.argmax(xt, axis=2).astype(jnp.int32)
    last = (V - 1) - jnp.argmax(xt[:, :, ::-1], axis=2).astype(jnp.int32)
    ii = jnp.arange(V, dtype=jnp.int32)[None, :]
    cnt = (first > ii).astype(jnp.int32) + (last > ii).astype(jnp.int32)
    start = jnp.cumsum(cnt, axis=1) - cnt
    kk = jnp.arange(V, dtype=jnp.int32)
    i_e = jnp.sum((start[:, :, None] <= kk[None, None, :]).astype(jnp.int32),
                  axis=1) - 1
    f_i = jnp.take_along_axis(first, i_e, axis=1)
    l_i = jnp.take_along_axis(last, i_e, axis=1)
    s_i = jnp.take_along_axis(start, i_e, axis=1)
    firstj = jnp.where(f_i > i_e, f_i, l_i)
    j_e = jnp.where(kk[None, :] == s_i, firstj, l_i)

    d = jnp.take_along_axis(x_tour_directed.reshape(B, V * V),
                            i_e * V + j_e, axis=1)
    U = jnp.where(d, i_e, j_e)
    Vv = jnp.where(d, j_e, i_e)

    boff = (jnp.arange(B, dtype=jnp.int32) * (V * V))[:, None]
    Uk1, Uk2 = U[:, RS_PAD], U[:, CS_PAD]
    Vk1, Vk2 = Vv[:, RS_PAD], Vv[:, CS_PAD]
    idx_all = jnp.stack([
        boff + Uk1 * V + Vk1,
        boff + Uk2 * V + Vk2,
        boff + Uk1 * V + Uk2,
        boff + Vk1 * V + Vk2,
    ]).reshape(R_ROWS)

    table = e.reshape(B * V * V, H)
    rows = _sc_gather(table, idx_all)
    quad = rows.reshape(NTAB, B, PP, H)

    noise = jax.random.gumbel(jax.random.key(42), (B, P), f32)
    noise = jnp.pad(noise, ((0, 0), (0, PP - P))).reshape(B, PP, 1)
    Wp, bp = p['pre_act']
    w1, b1 = p['act_hidden'][0]
    w2, b2 = p['act_hidden'][1]
    w3, b3 = p['act_hidden'][2]
    wo, bo = p['act_out']
    tab_spec = lambda t: pl.BlockSpec((1, 1, PP, H), lambda b, _t=t: (_t, b, 0, 0))
    act2, pi2 = pl.pallas_call(
        _mlp_body,
        grid=(B,),
        in_specs=[
            tab_spec(0), tab_spec(1), tab_spec(2), tab_spec(3),
            pl.BlockSpec((1, PP, 1), lambda b: (b, 0, 0)),
            _full((H, H)), _full((H, H)), _full((H, H)), _full((H, H)),
            _full((1, H)),
            _full((H, H)), _full((1, H)),
            _full((H, H)), _full((1, H)),
            _full((H, H)), _full((1, H)),
            _full((H, 1)), _full((1, 1)),
        ],
        out_specs=[pl.BlockSpec((1, 1, 1), lambda b: (b, 0, 0)),
                   pl.BlockSpec((1, 1, 1), lambda b: (b, 0, 0))],
        out_shape=[jax.ShapeDtypeStruct((B, 1, 1), jnp.int32),
                   jax.ShapeDtypeStruct((B, 1, 1), f32)],
    )(quad, quad, quad, quad, noise,
      Wp[0:H], Wp[H:2 * H], Wp[2 * H:3 * H], Wp[3 * H:4 * H], bp.reshape(1, H),
      w1, b1.reshape(1, H), w2, b2.reshape(1, H), w3, b3.reshape(1, H),
      wo, bo.reshape(1, 1))

    actions = act2[:, 0, 0]
    pi = pi2[:, 0, 0]

    k1 = jnp.asarray(RS_PAD)[actions]
    k2 = jnp.asarray(CS_PAD)[actions]
    barange = jnp.arange(B, dtype=jnp.int32)

    def edge_row(kidx):
        return jnp.stack([
            barange,
            jnp.take_along_axis(i_e, kidx[:, None], axis=1)[:, 0],
            jnp.take_along_axis(j_e, kidx[:, None], axis=1)[:, 0],
        ], axis=1)

    edges = jnp.stack([edge_row(k1), edge_row(k2)], axis=1)
    return edges, pi, actions


# speedup vs baseline: 6.1206x; 1.3605x over previous
import functools

import numpy as np
import jax
import jax.numpy as jnp
from jax import lax
from jax.experimental import pallas as pl
from jax.experimental.pallas import tpu as pltpu
from jax.experimental.pallas import tpu_sc as plsc

B, V, H = 8, 100, 128
HH = H // 2
P = V * (V - 1) // 2
PP = 4992
G_ROWS = 2 * B * PP
NW = 32
CHUNK = 96
ROWS_PER_W = G_ROWS // NW
NCHUNK = ROWS_PER_W // CHUNK
E_ROWS = 1024
E_PER_W = E_ROWS // NW
RT = 10

_RS, _CS = np.triu_indices(V, 1)
RS_PAD = np.concatenate([_RS, np.zeros(PP - P, np.int64)]).astype(np.int32)
CS_PAD = np.concatenate([_CS, np.zeros(PP - P, np.int64)]).astype(np.int32)
S1_ONEHOT = np.zeros((PP, V), np.float32)
S1_ONEHOT[np.arange(PP), RS_PAD] = 1.0
S2_ONEHOT = np.zeros((PP, V), np.float32)
S2_ONEHOT[np.arange(PP), CS_PAD] = 1.0


def _node1_body(coord_ref, wn_ref, vew_ref, veb_ref, unw_ref, unb_ref,
                vnw_ref, vnb_ref, x0_ref, vx_ref, ux_ref, vn_ref):
    cr = coord_ref[0]
    x = cr[:, 0:1] * wn_ref[0:1, :] + cr[:, 1:2] * wn_ref[1:2, :]
    x0_ref[0] = x
    vx_ref[0] = jnp.dot(x, vew_ref[...], preferred_element_type=jnp.float32) + veb_ref[...]
    ux_ref[0] = jnp.dot(x, unw_ref[...], preferred_element_type=jnp.float32) + unb_ref[...]
    vn_ref[0] = jnp.dot(x, vnw_ref[...], preferred_element_type=jnp.float32) + vnb_ref[...]


def _node_body(x_ref, vew_ref, veb_ref, unw_ref, unb_ref, vnw_ref, vnb_ref,
               vx_ref, ux_ref, vn_ref):
    x = x_ref[0]
    vx_ref[0] = jnp.dot(x, vew_ref[...], preferred_element_type=jnp.float32) + veb_ref[...]
    ux_ref[0] = jnp.dot(x, unw_ref[...], preferred_element_type=jnp.float32) + unb_ref[...]
    vn_ref[0] = jnp.dot(x, vnw_ref[...], preferred_element_type=jnp.float32) + vnb_ref[...]


def _edge_update(e_row, r, vxf, vxr_ref, uxr_ref, vnf, xr_ref,
                 uew_ref, ueb_ref, ge_ref, be_ref, gn_ref, bn_ref,
                 eo_ref, xo_ref):
    ue = jnp.dot(e_row, uew_ref[...], preferred_element_type=jnp.float32) + ueb_ref[...]
    e_tmp = ue + vxr_ref[0, r] + vxf
    gate = 1.0 / (1.0 + jnp.exp(-e_tmp))
    num = jnp.sum(gate * vnf, axis=0, keepdims=True)
    den = 1e-20 + jnp.sum(gate, axis=0, keepdims=True)
    x_tmp = uxr_ref[0, r] + num / den
    eo_ref[0, r] = e_row + jnp.maximum(e_tmp * ge_ref[...] + be_ref[...], 0.0)
    xo_ref[0, r] = xr_ref[0, r] + jnp.maximum(x_tmp * gn_ref[...] + bn_ref[...], 0.0)


def _edge1_body(vals_ref, tour_ref, best_ref, wev_ref, emb0_ref, emb1_ref,
                vxf_ref, vxr_ref, uxr_ref, vnf_ref, xr_ref,
                uew_ref, ueb_ref, ge_ref, be_ref, gn_ref, bn_ref,
                eo_ref, xo_ref):
    vxf = vxf_ref[0]
    vnf = vnf_ref[0]
    for r in range(RT):
        ev = vals_ref[0, r] * wev_ref[...]
        tags = (jnp.where(tour_ref[0, r] > 0, emb0_ref[1:2, :], emb0_ref[0:1, :])
                + jnp.where(best_ref[0, r] > 0, emb1_ref[1:2, :], emb1_ref[0:1, :]))
        e_row = jnp.concatenate([ev, tags], axis=-1)
        _edge_update(e_row, r, vxf, vxr_ref, uxr_ref, vnf, xr_ref,
                     uew_ref, ueb_ref, ge_ref, be_ref, gn_ref, bn_ref,
                     eo_ref, xo_ref)


def _edge_body(e_ref, vxf_ref, vxr_ref, uxr_ref, vnf_ref, xr_ref,
               uew_ref, ueb_ref, ge_ref, be_ref, gn_ref, bn_ref,
               eo_ref, xo_ref):
    vxf = vxf_ref[0]
    vnf = vnf_ref[0]
    for r in range(RT):
        _edge_update(e_ref[0, r], r, vxf, vxr_ref, uxr_ref, vnf, xr_ref,
                     uew_ref, ueb_ref, ge_ref, be_ref, gn_ref, bn_ref,
                     eo_ref, xo_ref)


def _sc_gather_body(table_hbm, idxg_hbm, idxe_hbm, outg_hbm, oute_hbm,
                    idx_v, idxe_v, rows0, rows1, rowse, sem0, sem1, seme):
    wid = lax.axis_index("s") * 2 + lax.axis_index("c")
    base = wid * ROWS_PER_W

    pltpu.sync_copy(idxg_hbm.at[wid], idx_v)
    pltpu.sync_copy(idxe_hbm.at[wid], idxe_v)
    pltpu.async_copy(table_hbm.at[idxe_v], rowse, seme)
    pltpu.async_copy(table_hbm.at[idx_v.at[0]], rows0, sem0)
    pltpu.async_copy(table_hbm.at[idx_v.at[1]], rows1, sem1)

    def step(s, carry):
        i0 = 2 * s
        i1 = i0 + 1
        pltpu.make_async_copy(table_hbm.at[idx_v.at[i0]], rows0, sem0).wait()
        pltpu.sync_copy(rows0, outg_hbm.at[pl.ds(base + i0 * CHUNK, CHUNK)])

        @pl.when(i0 + 2 < NCHUNK)
        def _():
            pltpu.async_copy(table_hbm.at[idx_v.at[i0 + 2]], rows0, sem0)

        pltpu.make_async_copy(table_hbm.at[idx_v.at[i1]], rows1, sem1).wait()
        pltpu.sync_copy(rows1, outg_hbm.at[pl.ds(base + i1 * CHUNK, CHUNK)])

        @pl.when(i1 + 2 < NCHUNK)
        def _():
            pltpu.async_copy(table_hbm.at[idx_v.at[i1 + 2]], rows1, sem1)

        return carry

    lax.fori_loop(0, NCHUNK // 2, step, 0)
    pltpu.make_async_copy(table_hbm.at[idxe_v], rowse, seme).wait()
    pltpu.sync_copy(rowse, oute_hbm.at[pl.ds(wid * E_PER_W, E_PER_W)])


_sc_gather = functools.partial(
    pl.kernel,
    out_type=[jax.ShapeDtypeStruct((G_ROWS, H), jnp.float32),
              jax.ShapeDtypeStruct((E_ROWS, H), jnp.float32)],
    mesh=plsc.VectorSubcoreMesh(core_axis_name="c", subcore_axis_name="s"),
    scratch_types=[
        pltpu.VMEM((NCHUNK, CHUNK), jnp.int32),
        pltpu.VMEM((E_PER_W,), jnp.int32),
        pltpu.VMEM((CHUNK, H), jnp.float32),
        pltpu.VMEM((CHUNK, H), jnp.float32),
        pltpu.VMEM((E_PER_W, H), jnp.float32),
        pltpu.SemaphoreType.DMA,
        pltpu.SemaphoreType.DMA,
        pltpu.SemaphoreType.DMA,
    ],
)(_sc_gather_body)


def _mlp_body(g1_ref, g2_ref, e1_ref, s1_ref, s2_ref, noise_ref,
              wa_ref, wb_ref, wc_ref, wd_ref, bp_ref,
              w1_ref, b1_ref, w2_ref, b2_ref, w3_ref, b3_ref,
              wo_ref, bo_ref,
              act_ref, pi_ref):
    f32 = jnp.float32
    a1 = jnp.dot(e1_ref[0], wa_ref[...], preferred_element_type=f32)
    a2 = jnp.dot(e1_ref[0], wb_ref[...], preferred_element_type=f32)
    h = (jnp.dot(s1_ref[...], a1, preferred_element_type=f32)
         + jnp.dot(s2_ref[...], a2, preferred_element_type=f32)
         + jnp.dot(g1_ref[0, 0], wc_ref[...], preferred_element_type=f32)
         + jnp.dot(g2_ref[0, 0], wd_ref[...], preferred_element_type=f32)
         + bp_ref[...])
    h = jnp.maximum(jnp.dot(h, w1_ref[...], preferred_element_type=f32) + b1_ref[...], 0.0)
    h = jnp.maximum(jnp.dot(h, w2_ref[...], preferred_element_type=f32) + b2_ref[...], 0.0)
    h = jnp.maximum(jnp.dot(h, w3_ref[...], preferred_element_type=f32) + b3_ref[...], 0.0)
    logits = jnp.dot(h, wo_ref[...], preferred_element_type=f32) + bo_ref[...]
    rowid = lax.broadcasted_iota(jnp.int32, (PP, 1), 0)
    logits = jnp.where(rowid < P, logits, f32(-1e30))
    z = logits + noise_ref[0]
    maxz = jnp.max(z)
    action = jnp.min(jnp.where(z >= maxz, rowid, jnp.int32(PP)))
    m = jnp.max(logits)
    lse = m + jnp.log(jnp.sum(jnp.exp(logits - m)))
    logit_a = jnp.sum(jnp.where(rowid == action, logits, 0.0))
    act_ref[0] = action[None, None]
    pi_ref[0] = (logit_a - lse)[None, None]


def _full(shape):
    nd = len(shape)
    return pl.BlockSpec(shape, lambda *a: (0,) * nd)


def kernel(x_edges, x_edges_values, x_nodes_coord, x_tour, x_best_tour,
           x_tour_directed, params):
    p = params
    f32 = jnp.float32
    cbn = np.float32(1.0 / np.sqrt(1.0 + 1e-5))
    xt = x_tour.astype(jnp.int32)
    xb = x_best_tour.astype(jnp.int32)

    vals4 = x_edges_values.reshape(B, V, V, 1)
    t4 = xt.reshape(B, V, V, 1)
    b4 = xb.reshape(B, V, V, 1)
    wev = p['W_evals'].reshape(1, HH)

    node_w_specs = [_full((H, H)), _full((1, H)),
                    _full((H, H)), _full((1, H)),
                    _full((H, H)), _full((1, H))]
    bvh_spec = pl.BlockSpec((1, V, H), lambda b: (b, 0, 0))

    e = None
    x = None
    for li, lp in enumerate(p['layers']):
        node_w = (lp['Ve'][0], lp['Ve'][1].reshape(1, H),
                  lp['Un'][0], lp['Un'][1].reshape(1, H),
                  lp['Vn'][0], lp['Vn'][1].reshape(1, H))
        if li == 0:
            x, vx, ux, vn = pl.pallas_call(
                _node1_body,
                grid=(B,),
                in_specs=[pl.BlockSpec((1, V, 2), lambda b: (b, 0, 0)),
                          _full((2, H))] + node_w_specs,
                out_specs=[bvh_spec] * 4,
                out_shape=[jax.ShapeDtypeStruct((B, V, H), f32)] * 4,
            )(x_nodes_coord, p['W_nodes'], *node_w)
        else:
            vx, ux, vn = pl.pallas_call(
                _node_body,
                grid=(B,),
                in_specs=[bvh_spec] + node_w_specs,
                out_specs=[bvh_spec] * 3,
                out_shape=[jax.ShapeDtypeStruct((B, V, H), f32)] * 3,
            )(x, *node_w)

        ge = (lp['bn_e'][0] * cbn).reshape(1, H)
        be = lp['bn_e'][1].reshape(1, H)
        gn = (lp['bn_n'][0] * cbn).reshape(1, H)
        bn = lp['bn_n'][1].reshape(1, H)
        row_spec = pl.BlockSpec((1, RT, 1, H), lambda b, i: (b, i, 0, 0))
        shared_specs = [
            pl.BlockSpec((1, V, H), lambda b, i: (b, 0, 0)),
            row_spec,
            row_spec,
            pl.BlockSpec((1, V, H), lambda b, i: (b, 0, 0)),
            row_spec,
            pl.BlockSpec((H, H), lambda b, i: (0, 0)),
            pl.BlockSpec((1, H), lambda b, i: (0, 0)),
            pl.BlockSpec((1, H), lambda b, i: (0, 0)),
            pl.BlockSpec((1, H), lambda b, i: (0, 0)),
            pl.BlockSpec((1, H), lambda b, i: (0, 0)),
            pl.BlockSpec((1, H), lambda b, i: (0, 0)),
        ]
        shared_args = (vx, vx.reshape(B, V, 1, H), ux.reshape(B, V, 1, H),
                       vn, x.reshape(B, V, 1, H),
                       lp['Ue'][0], lp['Ue'][1].reshape(1, H), ge, be, gn, bn)
        out_specs = [pl.BlockSpec((1, RT, V, H), lambda b, i: (b, i, 0, 0)),
                     row_spec]
        out_shape = [jax.ShapeDtypeStruct((B, V, V, H), f32),
                     jax.ShapeDtypeStruct((B, V, 1, H), f32)]
        if li == 0:
            e, x4 = pl.pallas_call(
                _edge1_body,
                grid=(B, V // RT),
                in_specs=[
                    pl.BlockSpec((1, RT, V, 1), lambda b, i: (b, i, 0, 0)),
                    pl.BlockSpec((1, RT, V, 1), lambda b, i: (b, i, 0, 0)),
                    pl.BlockSpec((1, RT, V, 1), lambda b, i: (b, i, 0, 0)),
                    _full((1, HH)), _full((3, HH)), _full((3, HH)),
                ] + shared_specs,
                out_specs=out_specs,
                out_shape=out_shape,
            )(vals4, t4, b4, wev, p['emb0'], p['emb1'], *shared_args)
        else:
            e, x4 = pl.pallas_call(
                _edge_body,
                grid=(B, V // RT),
                in_specs=[pl.BlockSpec((1, RT, V, H), lambda b, i: (b, i, 0, 0))]
                + shared_specs,
                out_specs=out_specs,
                out_shape=out_shape,
            )(e, *shared_args)
        x = x4.reshape(B, V, H)

    first = jnp.argmax(xt, axis=2).astype(jnp.int32)
    last = (V - 1) - jnp.argmax(xt[:, :, ::-1], axis=2).astype(jnp.int32)
    ii = jnp.arange(V, dtype=jnp.int32)[None, :]
    cnt = (first > ii).astype(jnp.int32) + (last > ii).astype(jnp.int32)
    start = jnp.cumsum(cnt, axis=1) - cnt
    kk = jnp.arange(V, dtype=jnp.int32)
    i_e = jnp.sum((start[:, :, None] <= kk[None, None, :]).astype(jnp.int32),
                  axis=1) - 1
    f_i = jnp.take_along_axis(first, i_e, axis=1)
    l_i = jnp.take_along_axis(last, i_e, axis=1)
    s_i = jnp.take_along_axis(start, i_e, axis=1)
    firstj = jnp.where(f_i > i_e, f_i, l_i)
    j_e = jnp.where(kk[None, :] == s_i, firstj, l_i)

    d = jnp.take_along_axis(x_tour_directed.reshape(B, V * V),
                            i_e * V + j_e, axis=1)
    U = jnp.where(d, i_e, j_e)
    Vv = jnp.where(d, j_e, i_e)

    boff = (jnp.arange(B, dtype=jnp.int32) * (V * V))[:, None]
    Uk1, Uk2 = U[:, RS_PAD], U[:, CS_PAD]
    Vk1, Vk2 = Vv[:, RS_PAD], Vv[:, CS_PAD]
    idx_g = jnp.stack([
        boff + Uk1 * V + Uk2,
        boff + Vk1 * V + Vk2,
    ]).reshape(NW, NCHUNK, CHUNK)
    idx_e = jnp.pad((boff + U * V + Vv).reshape(B * V),
                    (0, E_ROWS - B * V)).reshape(NW, E_PER_W)

    table = e.reshape(B * V * V, H)
    rows_g, rows_e = _sc_gather(table, idx_g, idx_e)
    quad = rows_g.reshape(2, B, PP, H)
    e1 = rows_e[:B * V].reshape(B, V, H)

    noise = jax.random.gumbel(jax.random.key(42), (B, P), f32)
    noise = jnp.pad(noise, ((0, 0), (0, PP - P))).reshape(B, PP, 1)
    Wp, bp = p['pre_act']
    w1, b1 = p['act_hidden'][0]
    w2, b2 = p['act_hidden'][1]
    w3, b3 = p['act_hidden'][2]
    wo, bo = p['act_out']
    tab_spec = lambda t: pl.BlockSpec((1, 1, PP, H), lambda b, _t=t: (_t, b, 0, 0))
    act2, pi2 = pl.pallas_call(
        _mlp_body,
        grid=(B,),
        in_specs=[
            tab_spec(0), tab_spec(1),
            pl.BlockSpec((1, V, H), lambda b: (b, 0, 0)),
            _full((PP, V)), _full((PP, V)),
            pl.BlockSpec((1, PP, 1), lambda b: (b, 0, 0)),
            _full((H, H)), _full((H, H)), _full((H, H)), _full((H, H)),
            _full((1, H)),
            _full((H, H)), _full((1, H)),
            _full((H, H)), _full((1, H)),
            _full((H, H)), _full((1, H)),
            _full((H, 1)), _full((1, 1)),
        ],
        out_specs=[pl.BlockSpec((1, 1, 1), lambda b: (b, 0, 0)),
                   pl.BlockSpec((1, 1, 1), lambda b: (b, 0, 0))],
        out_shape=[jax.ShapeDtypeStruct((B, 1, 1), jnp.int32),
                   jax.ShapeDtypeStruct((B, 1, 1), f32)],
    )(quad, quad, e1, jnp.asarray(S1_ONEHOT), jnp.asarray(S2_ONEHOT), noise,
      Wp[0:H], Wp[H:2 * H], Wp[2 * H:3 * H], Wp[3 * H:4 * H], bp.reshape(1, H),
      w1, b1.reshape(1, H), w2, b2.reshape(1, H), w3, b3.reshape(1, H),
      wo, bo.reshape(1, 1))

    actions = act2[:, 0, 0]
    pi = pi2[:, 0, 0]

    k1 = jnp.asarray(RS_PAD)[actions]
    k2 = jnp.asarray(CS_PAD)[actions]
    barange = jnp.arange(B, dtype=jnp.int32)

    def edge_row(kidx):
        return jnp.stack([
            barange,
            jnp.take_along_axis(i_e, kidx[:, None], axis=1)[:, 0],
            jnp.take_along_axis(j_e, kidx[:, None], axis=1)[:, 0],
        ], axis=1)

    edges = jnp.stack([edge_row(k1), edge_row(k2)], axis=1)
    return edges, pi, actions


# speedup vs baseline: 6.8720x; 1.1228x over previous
import functools

import numpy as np
import jax
import jax.numpy as jnp
from jax import lax
from jax.experimental import pallas as pl
from jax.experimental.pallas import tpu as pltpu
from jax.experimental.pallas import tpu_sc as plsc

B, V, H = 8, 100, 128
HH = H // 2
P = V * (V - 1) // 2
PP = 4992
G_ROWS = 2 * B * PP
NW = 32
CHUNK = 96
ROWS_PER_W = G_ROWS // NW
NCHUNK = ROWS_PER_W // CHUNK
E_ROWS = 1024
E_PER_W = E_ROWS // NW
RT = 20

_RS, _CS = np.triu_indices(V, 1)
RS_PAD = np.concatenate([_RS, np.zeros(PP - P, np.int64)]).astype(np.int32)
CS_PAD = np.concatenate([_CS, np.zeros(PP - P, np.int64)]).astype(np.int32)
S1_ONEHOT = np.zeros((PP, V), np.float32)
S1_ONEHOT[np.arange(PP), RS_PAD] = 1.0
S2_ONEHOT = np.zeros((PP, V), np.float32)
S2_ONEHOT[np.arange(PP), CS_PAD] = 1.0


def _node1_body(coord_ref, wn_ref, vew_ref, veb_ref, unw_ref, unb_ref,
                vnw_ref, vnb_ref, x0_ref, vx_ref, ux_ref, vn_ref):
    cr = coord_ref[0]
    x = cr[:, 0:1] * wn_ref[0:1, :] + cr[:, 1:2] * wn_ref[1:2, :]
    x0_ref[0] = x
    vx_ref[0] = jnp.dot(x, vew_ref[...], preferred_element_type=jnp.float32) + veb_ref[...]
    ux_ref[0] = jnp.dot(x, unw_ref[...], preferred_element_type=jnp.float32) + unb_ref[...]
    vn_ref[0] = jnp.dot(x, vnw_ref[...], preferred_element_type=jnp.float32) + vnb_ref[...]


def _node_body(x_ref, vew_ref, veb_ref, unw_ref, unb_ref, vnw_ref, vnb_ref,
               vx_ref, ux_ref, vn_ref):
    x = x_ref[0]
    vx_ref[0] = jnp.dot(x, vew_ref[...], preferred_element_type=jnp.float32) + veb_ref[...]
    ux_ref[0] = jnp.dot(x, unw_ref[...], preferred_element_type=jnp.float32) + unb_ref[...]
    vn_ref[0] = jnp.dot(x, vnw_ref[...], preferred_element_type=jnp.float32) + vnb_ref[...]


def _node_last_body(x_ref, vew_ref, veb_ref, vx_ref):
    x = x_ref[0]
    vx_ref[0] = jnp.dot(x, vew_ref[...], preferred_element_type=jnp.float32) + veb_ref[...]


def _edge_update(e_row, r, vxf, vxr_ref, uxr_ref, vnf, xr_ref,
                 uew_ref, ueb_ref, ge_ref, be_ref, gn_ref, bn_ref,
                 eo_ref, xo_ref):
    ue = jnp.dot(e_row, uew_ref[...], preferred_element_type=jnp.float32) + ueb_ref[...]
    e_tmp = ue + vxr_ref[0, r] + vxf
    gate = 1.0 / (1.0 + jnp.exp(-e_tmp))
    num = jnp.sum(gate * vnf, axis=0, keepdims=True)
    den = 1e-20 + jnp.sum(gate, axis=0, keepdims=True)
    x_tmp = uxr_ref[0, r] + num / den
    eo_ref[0, r] = e_row + jnp.maximum(e_tmp * ge_ref[...] + be_ref[...], 0.0)
    xo_ref[0, r] = xr_ref[0, r] + jnp.maximum(x_tmp * gn_ref[...] + bn_ref[...], 0.0)


def _edge1_body(vals_ref, tour_ref, best_ref, wev_ref, emb0_ref, emb1_ref,
                vxf_ref, vxr_ref, uxr_ref, vnf_ref, xr_ref,
                uew_ref, ueb_ref, ge_ref, be_ref, gn_ref, bn_ref,
                eo_ref, xo_ref):
    vxf = vxf_ref[0]
    vnf = vnf_ref[0]
    for r in range(RT):
        ev = vals_ref[0, r] * wev_ref[...]
        tags = (jnp.where(tour_ref[0, r] > 0, emb0_ref[1:2, :], emb0_ref[0:1, :])
                + jnp.where(best_ref[0, r] > 0, emb1_ref[1:2, :], emb1_ref[0:1, :]))
        e_row = jnp.concatenate([ev, tags], axis=-1)
        _edge_update(e_row, r, vxf, vxr_ref, uxr_ref, vnf, xr_ref,
                     uew_ref, ueb_ref, ge_ref, be_ref, gn_ref, bn_ref,
                     eo_ref, xo_ref)


def _edge_body(e_ref, vxf_ref, vxr_ref, uxr_ref, vnf_ref, xr_ref,
               uew_ref, ueb_ref, ge_ref, be_ref, gn_ref, bn_ref,
               eo_ref, xo_ref):
    vxf = vxf_ref[0]
    vnf = vnf_ref[0]
    for r in range(RT):
        _edge_update(e_ref[0, r], r, vxf, vxr_ref, uxr_ref, vnf, xr_ref,
                     uew_ref, ueb_ref, ge_ref, be_ref, gn_ref, bn_ref,
                     eo_ref, xo_ref)


def _edge_last_body(e_ref, vxf_ref, vxr_ref,
                    uew_ref, ueb_ref, ge_ref, be_ref, eo_ref):
    vxf = vxf_ref[0]
    for r in range(RT):
        e_row = e_ref[0, r]
        ue = jnp.dot(e_row, uew_ref[...], preferred_element_type=jnp.float32) + ueb_ref[...]
        e_tmp = ue + vxr_ref[0, r] + vxf
        eo_ref[0, r] = e_row + jnp.maximum(e_tmp * ge_ref[...] + be_ref[...], 0.0)


def _sc_gather_body(table_hbm, idxg_hbm, idxe_hbm, outg_hbm, oute_hbm,
                    idx_v, idxe_v, rows0, rows1, rowse, sem0, sem1, seme):
    wid = lax.axis_index("s") * 2 + lax.axis_index("c")
    base = wid * ROWS_PER_W

    pltpu.sync_copy(idxg_hbm.at[wid], idx_v)
    pltpu.sync_copy(idxe_hbm.at[wid], idxe_v)
    pltpu.async_copy(table_hbm.at[idxe_v], rowse, seme)
    pltpu.async_copy(table_hbm.at[idx_v.at[0]], rows0, sem0)
    pltpu.async_copy(table_hbm.at[idx_v.at[1]], rows1, sem1)

    def step(s, carry):
        i0 = 2 * s
        i1 = i0 + 1
        pltpu.make_async_copy(table_hbm.at[idx_v.at[i0]], rows0, sem0).wait()
        pltpu.sync_copy(rows0, outg_hbm.at[pl.ds(base + i0 * CHUNK, CHUNK)])

        @pl.when(i0 + 2 < NCHUNK)
        def _():
            pltpu.async_copy(table_hbm.at[idx_v.at[i0 + 2]], rows0, sem0)

        pltpu.make_async_copy(table_hbm.at[idx_v.at[i1]], rows1, sem1).wait()
        pltpu.sync_copy(rows1, outg_hbm.at[pl.ds(base + i1 * CHUNK, CHUNK)])

        @pl.when(i1 + 2 < NCHUNK)
        def _():
            pltpu.async_copy(table_hbm.at[idx_v.at[i1 + 2]], rows1, sem1)

        return carry

    lax.fori_loop(0, NCHUNK // 2, step, 0)
    pltpu.make_async_copy(table_hbm.at[idxe_v], rowse, seme).wait()
    pltpu.sync_copy(rowse, oute_hbm.at[pl.ds(wid * E_PER_W, E_PER_W)])


_sc_gather = functools.partial(
    pl.kernel,
    out_type=[jax.ShapeDtypeStruct((G_ROWS, H), jnp.float32),
              jax.ShapeDtypeStruct((E_ROWS, H), jnp.float32)],
    mesh=plsc.VectorSubcoreMesh(core_axis_name="c", subcore_axis_name="s"),
    scratch_types=[
        pltpu.VMEM((NCHUNK, CHUNK), jnp.int32),
        pltpu.VMEM((E_PER_W,), jnp.int32),
        pltpu.VMEM((CHUNK, H), jnp.float32),
        pltpu.VMEM((CHUNK, H), jnp.float32),
        pltpu.VMEM((E_PER_W, H), jnp.float32),
        pltpu.SemaphoreType.DMA,
        pltpu.SemaphoreType.DMA,
        pltpu.SemaphoreType.DMA,
    ],
)(_sc_gather_body)


def _mlp_body(g1_ref, g2_ref, e1_ref, s1_ref, s2_ref, noise_ref,
              wa_ref, wb_ref, wc_ref, wd_ref, bp_ref,
              w1_ref, b1_ref, w2_ref, b2_ref, w3_ref, b3_ref,
              wo_ref, bo_ref,
              act_ref, pi_ref):
    f32 = jnp.float32
    a1 = jnp.dot(e1_ref[0], wa_ref[...], preferred_element_type=f32)
    a2 = jnp.dot(e1_ref[0], wb_ref[...], preferred_element_type=f32)
    h = (jnp.dot(s1_ref[...], a1, preferred_element_type=f32)
         + jnp.dot(s2_ref[...], a2, preferred_element_type=f32)
         + jnp.dot(g1_ref[0, 0], wc_ref[...], preferred_element_type=f32)
         + jnp.dot(g2_ref[0, 0], wd_ref[...], preferred_element_type=f32)
         + bp_ref[...])
    h = jnp.maximum(jnp.dot(h, w1_ref[...], preferred_element_type=f32) + b1_ref[...], 0.0)
    h = jnp.maximum(jnp.dot(h, w2_ref[...], preferred_element_type=f32) + b2_ref[...], 0.0)
    h = jnp.maximum(jnp.dot(h, w3_ref[...], preferred_element_type=f32) + b3_ref[...], 0.0)
    logits = jnp.dot(h, wo_ref[...], preferred_element_type=f32) + bo_ref[...]
    rowid = lax.broadcasted_iota(jnp.int32, (PP, 1), 0)
    logits = jnp.where(rowid < P, logits, f32(-1e30))
    z = logits + noise_ref[0]
    maxz = jnp.max(z)
    action = jnp.min(jnp.where(z >= maxz, rowid, jnp.int32(PP)))
    m = jnp.max(logits)
    lse = m + jnp.log(jnp.sum(jnp.exp(logits - m)))
    logit_a = jnp.sum(jnp.where(rowid == action, logits, 0.0))
    act_ref[0] = action[None, None]
    pi_ref[0] = (logit_a - lse)[None, None]


def _full(shape):
    nd = len(shape)
    return pl.BlockSpec(shape, lambda *a: (0,) * nd)


def kernel(x_edges, x_edges_values, x_nodes_coord, x_tour, x_best_tour,
           x_tour_directed, params):
    p = params
    f32 = jnp.float32
    cbn = np.float32(1.0 / np.sqrt(1.0 + 1e-5))
    xt = x_tour.astype(jnp.int32)
    xb = x_best_tour.astype(jnp.int32)

    vals4 = x_edges_values.reshape(B, V, V, 1)
    t4 = xt.reshape(B, V, V, 1)
    b4 = xb.reshape(B, V, V, 1)
    wev = p['W_evals'].reshape(1, HH)

    node_w_specs = [_full((H, H)), _full((1, H)),
                    _full((H, H)), _full((1, H)),
                    _full((H, H)), _full((1, H))]
    bvh_spec = pl.BlockSpec((1, V, H), lambda b: (b, 0, 0))

    par1 = pltpu.CompilerParams(dimension_semantics=("parallel",))
    par2 = pltpu.CompilerParams(dimension_semantics=("parallel", "parallel"))
    e = None
    x = None
    for li, lp in enumerate(p['layers']):
        last = li == len(p['layers']) - 1
        if li == 0:
            node_w = (lp['Ve'][0], lp['Ve'][1].reshape(1, H),
                      lp['Un'][0], lp['Un'][1].reshape(1, H),
                      lp['Vn'][0], lp['Vn'][1].reshape(1, H))
            x, vx, ux, vn = pl.pallas_call(
                _node1_body,
                grid=(B,),
                in_specs=[pl.BlockSpec((1, V, 2), lambda b: (b, 0, 0)),
                          _full((2, H))] + node_w_specs,
                out_specs=[bvh_spec] * 4,
                out_shape=[jax.ShapeDtypeStruct((B, V, H), f32)] * 4,
                compiler_params=par1,
            )(x_nodes_coord, p['W_nodes'], *node_w)
        elif not last:
            node_w = (lp['Ve'][0], lp['Ve'][1].reshape(1, H),
                      lp['Un'][0], lp['Un'][1].reshape(1, H),
                      lp['Vn'][0], lp['Vn'][1].reshape(1, H))
            vx, ux, vn = pl.pallas_call(
                _node_body,
                grid=(B,),
                in_specs=[bvh_spec] + node_w_specs,
                out_specs=[bvh_spec] * 3,
                out_shape=[jax.ShapeDtypeStruct((B, V, H), f32)] * 3,
                compiler_params=par1,
            )(x, *node_w)
        else:
            vx = pl.pallas_call(
                _node_last_body,
                grid=(B,),
                in_specs=[bvh_spec, _full((H, H)), _full((1, H))],
                out_specs=bvh_spec,
                out_shape=jax.ShapeDtypeStruct((B, V, H), f32),
                compiler_params=par1,
            )(x, lp['Ve'][0], lp['Ve'][1].reshape(1, H))

        ge = (lp['bn_e'][0] * cbn).reshape(1, H)
        be = lp['bn_e'][1].reshape(1, H)
        row_spec = pl.BlockSpec((1, RT, 1, H), lambda b, i: (b, i, 0, 0))
        w_spec = pl.BlockSpec((H, H), lambda b, i: (0, 0))
        h_spec = pl.BlockSpec((1, H), lambda b, i: (0, 0))
        vxf_spec = pl.BlockSpec((1, V, H), lambda b, i: (b, 0, 0))
        e_spec = pl.BlockSpec((1, RT, V, H), lambda b, i: (b, i, 0, 0))
        if last:
            e = pl.pallas_call(
                _edge_last_body,
                grid=(B, V // RT),
                in_specs=[e_spec, vxf_spec, row_spec,
                          w_spec, h_spec, h_spec, h_spec],
                out_specs=e_spec,
                out_shape=jax.ShapeDtypeStruct((B, V, V, H), f32),
                compiler_params=par2,
            )(e, vx, vx.reshape(B, V, 1, H),
              lp['Ue'][0], lp['Ue'][1].reshape(1, H), ge, be)
            break

        gn = (lp['bn_n'][0] * cbn).reshape(1, H)
        bn = lp['bn_n'][1].reshape(1, H)
        shared_specs = [
            vxf_spec,
            row_spec,
            row_spec,
            pl.BlockSpec((1, V, H), lambda b, i: (b, 0, 0)),
            row_spec,
            w_spec, h_spec, h_spec, h_spec, h_spec, h_spec,
        ]
        shared_args = (vx, vx.reshape(B, V, 1, H), ux.reshape(B, V, 1, H),
                       vn, x.reshape(B, V, 1, H),
                       lp['Ue'][0], lp['Ue'][1].reshape(1, H), ge, be, gn, bn)
        out_specs = [e_spec, row_spec]
        out_shape = [jax.ShapeDtypeStruct((B, V, V, H), f32),
                     jax.ShapeDtypeStruct((B, V, 1, H), f32)]
        if li == 0:
            e, x4 = pl.pallas_call(
                _edge1_body,
                grid=(B, V // RT),
                in_specs=[
                    pl.BlockSpec((1, RT, V, 1), lambda b, i: (b, i, 0, 0)),
                    pl.BlockSpec((1, RT, V, 1), lambda b, i: (b, i, 0, 0)),
                    pl.BlockSpec((1, RT, V, 1), lambda b, i: (b, i, 0, 0)),
                    _full((1, HH)), _full((3, HH)), _full((3, HH)),
                ] + shared_specs,
                out_specs=out_specs,
                out_shape=out_shape,
                compiler_params=par2,
            )(vals4, t4, b4, wev, p['emb0'], p['emb1'], *shared_args)
        else:
            e, x4 = pl.pallas_call(
                _edge_body,
                grid=(B, V // RT),
                in_specs=[e_spec] + shared_specs,
                out_specs=out_specs,
                out_shape=out_shape,
                compiler_params=par2,
            )(e, *shared_args)
        x = x4.reshape(B, V, H)

    first = jnp.argmax(xt, axis=2).astype(jnp.int32)
    last = (V - 1) - jnp.argmax(xt[:, :, ::-1], axis=2).astype(jnp.int32)
    ii = jnp.arange(V, dtype=jnp.int32)[None, :]
    cnt = (first > ii).astype(jnp.int32) + (last > ii).astype(jnp.int32)
    start = jnp.cumsum(cnt, axis=1) - cnt
    kk = jnp.arange(V, dtype=jnp.int32)
    i_e = jnp.sum((start[:, :, None] <= kk[None, None, :]).astype(jnp.int32),
                  axis=1) - 1
    f_i = jnp.take_along_axis(first, i_e, axis=1)
    l_i = jnp.take_along_axis(last, i_e, axis=1)
    s_i = jnp.take_along_axis(start, i_e, axis=1)
    firstj = jnp.where(f_i > i_e, f_i, l_i)
    j_e = jnp.where(kk[None, :] == s_i, firstj, l_i)

    d = jnp.take_along_axis(x_tour_directed.reshape(B, V * V),
                            i_e * V + j_e, axis=1)
    U = jnp.where(d, i_e, j_e)
    Vv = jnp.where(d, j_e, i_e)

    boff = (jnp.arange(B, dtype=jnp.int32) * (V * V))[:, None]
    Uk1, Uk2 = U[:, RS_PAD], U[:, CS_PAD]
    Vk1, Vk2 = Vv[:, RS_PAD], Vv[:, CS_PAD]
    idx_g = jnp.stack([
        boff + Uk1 * V + Uk2,
        boff + Vk1 * V + Vk2,
    ]).reshape(NW, NCHUNK, CHUNK)
    idx_e = jnp.pad((boff + U * V + Vv).reshape(B * V),
                    (0, E_ROWS - B * V)).reshape(NW, E_PER_W)

    table = e.reshape(B * V * V, H)
    rows_g, rows_e = _sc_gather(table, idx_g, idx_e)
    quad = rows_g.reshape(2, B, PP, H)
    e1 = rows_e[:B * V].reshape(B, V, H)

    noise = jax.random.gumbel(jax.random.key(42), (B, P), f32)
    noise = jnp.pad(noise, ((0, 0), (0, PP - P))).reshape(B, PP, 1)
    Wp, bp = p['pre_act']
    w1, b1 = p['act_hidden'][0]
    w2, b2 = p['act_hidden'][1]
    w3, b3 = p['act_hidden'][2]
    wo, bo = p['act_out']
    tab_spec = lambda t: pl.BlockSpec((1, 1, PP, H), lambda b, _t=t: (_t, b, 0, 0))
    act2, pi2 = pl.pallas_call(
        _mlp_body,
        grid=(B,),
        in_specs=[
            tab_spec(0), tab_spec(1),
            pl.BlockSpec((1, V, H), lambda b: (b, 0, 0)),
            _full((PP, V)), _full((PP, V)),
            pl.BlockSpec((1, PP, 1), lambda b: (b, 0, 0)),
            _full((H, H)), _full((H, H)), _full((H, H)), _full((H, H)),
            _full((1, H)),
            _full((H, H)), _full((1, H)),
            _full((H, H)), _full((1, H)),
            _full((H, H)), _full((1, H)),
            _full((H, 1)), _full((1, 1)),
        ],
        out_specs=[pl.BlockSpec((1, 1, 1), lambda b: (b, 0, 0)),
                   pl.BlockSpec((1, 1, 1), lambda b: (b, 0, 0))],
        out_shape=[jax.ShapeDtypeStruct((B, 1, 1), jnp.int32),
                   jax.ShapeDtypeStruct((B, 1, 1), f32)],
    )(quad, quad, e1, jnp.asarray(S1_ONEHOT), jnp.asarray(S2_ONEHOT), noise,
      Wp[0:H], Wp[H:2 * H], Wp[2 * H:3 * H], Wp[3 * H:4 * H], bp.reshape(1, H),
      w1, b1.reshape(1, H), w2, b2.reshape(1, H), w3, b3.reshape(1, H),
      wo, bo.reshape(1, 1))

    actions = act2[:, 0, 0]
    pi = pi2[:, 0, 0]

    k1 = jnp.asarray(RS_PAD)[actions]
    k2 = jnp.asarray(CS_PAD)[actions]
    barange = jnp.arange(B, dtype=jnp.int32)

    def edge_row(kidx):
        return jnp.stack([
            barange,
            jnp.take_along_axis(i_e, kidx[:, None], axis=1)[:, 0],
            jnp.take_along_axis(j_e, kidx[:, None], axis=1)[:, 0],
        ], axis=1)

    edges = jnp.stack([edge_row(k1), edge_row(k2)], axis=1)
    return edges, pi, actions


# speedup vs baseline: 7.1631x; 1.0424x over previous
import functools

import numpy as np
import jax
import jax.numpy as jnp
from jax import lax
from jax.experimental import pallas as pl
from jax.experimental.pallas import tpu as pltpu
from jax.experimental.pallas import tpu_sc as plsc

B, V, H = 8, 100, 128
HH = H // 2
P = V * (V - 1) // 2
PP = 4992
G_ROWS = 2 * B * PP
NW = 32
CHUNK = 96
ROWS_PER_W = G_ROWS // NW
NCHUNK = ROWS_PER_W // CHUNK
E_ROWS = 1024
E_PER_W = E_ROWS // NW
RT = 20

_RS, _CS = np.triu_indices(V, 1)
RS_PAD = np.concatenate([_RS, np.zeros(PP - P, np.int64)]).astype(np.int32)
CS_PAD = np.concatenate([_CS, np.zeros(PP - P, np.int64)]).astype(np.int32)
S1_ONEHOT = np.zeros((PP, V), np.float32)
S1_ONEHOT[np.arange(PP), RS_PAD] = 1.0
S2_ONEHOT = np.zeros((PP, V), np.float32)
S2_ONEHOT[np.arange(PP), CS_PAD] = 1.0


def _edge_update(e_row, r, vxf, vx_tile, ux_tile, vnf, x_tile,
                 uew_ref, ueb_ref, ge_ref, be_ref, gn_ref, bn_ref,
                 eo_ref, xo_ref):
    ue = jnp.dot(e_row, uew_ref[...], preferred_element_type=jnp.float32) + ueb_ref[...]
    e_tmp = ue + vx_tile[r:r + 1] + vxf
    gate = 1.0 / (1.0 + jnp.exp(-e_tmp))
    num = jnp.sum(gate * vnf, axis=0, keepdims=True)
    den = 1e-20 + jnp.sum(gate, axis=0, keepdims=True)
    x_tmp = ux_tile[r:r + 1] + num / den
    eo_ref[0, r] = e_row + jnp.maximum(e_tmp * ge_ref[...] + be_ref[...], 0.0)
    xo_ref[0, r] = x_tile[r:r + 1] + jnp.maximum(x_tmp * gn_ref[...] + bn_ref[...], 0.0)


def _node_products(xf, xt, vew_ref, veb_ref, unw_ref, unb_ref,
                   vnw_ref, vnb_ref):
    f32 = jnp.float32
    vxf = jnp.dot(xf, vew_ref[...], preferred_element_type=f32) + veb_ref[...]
    vnf = jnp.dot(xf, vnw_ref[...], preferred_element_type=f32) + vnb_ref[...]
    vx_tile = jnp.dot(xt, vew_ref[...], preferred_element_type=f32) + veb_ref[...]
    ux_tile = jnp.dot(xt, unw_ref[...], preferred_element_type=f32) + unb_ref[...]
    return vxf, vnf, vx_tile, ux_tile


def _edge1_body(vals_ref, tour_ref, best_ref, wev_ref, emb0_ref, emb1_ref,
                coordf_ref, coordt_ref, wn_ref,
                vew_ref, veb_ref, unw_ref, unb_ref, vnw_ref, vnb_ref,
                uew_ref, ueb_ref, ge_ref, be_ref, gn_ref, bn_ref,
                eo_ref, xo_ref):
    cf = coordf_ref[0]
    xf = cf[:, 0:1] * wn_ref[0:1, :] + cf[:, 1:2] * wn_ref[1:2, :]
    ct = coordt_ref[0].reshape(RT, 2)
    xt = ct[:, 0:1] * wn_ref[0:1, :] + ct[:, 1:2] * wn_ref[1:2, :]
    vxf, vnf, vx_tile, ux_tile = _node_products(
        xf, xt, vew_ref, veb_ref, unw_ref, unb_ref, vnw_ref, vnb_ref)
    for r in range(RT):
        ev = vals_ref[0, r] * wev_ref[...]
        tags = (jnp.where(tour_ref[0, r] > 0, emb0_ref[1:2, :], emb0_ref[0:1, :])
                + jnp.where(best_ref[0, r] > 0, emb1_ref[1:2, :], emb1_ref[0:1, :]))
        e_row = jnp.concatenate([ev, tags], axis=-1)
        _edge_update(e_row, r, vxf, vx_tile, ux_tile, vnf, xt,
                     uew_ref, ueb_ref, ge_ref, be_ref, gn_ref, bn_ref,
                     eo_ref, xo_ref)


def _edge_body(e_ref, xf_ref, xt_ref,
               vew_ref, veb_ref, unw_ref, unb_ref, vnw_ref, vnb_ref,
               uew_ref, ueb_ref, ge_ref, be_ref, gn_ref, bn_ref,
               eo_ref, xo_ref):
    xf = xf_ref[0]
    xt = xt_ref[0].reshape(RT, H)
    vxf, vnf, vx_tile, ux_tile = _node_products(
        xf, xt, vew_ref, veb_ref, unw_ref, unb_ref, vnw_ref, vnb_ref)
    for r in range(RT):
        _edge_update(e_ref[0, r], r, vxf, vx_tile, ux_tile, vnf, xt,
                     uew_ref, ueb_ref, ge_ref, be_ref, gn_ref, bn_ref,
                     eo_ref, xo_ref)


def _edge_last_body(e_ref, xf_ref, xt_ref, vew_ref, veb_ref,
                    uew_ref, ueb_ref, ge_ref, be_ref, eo_ref):
    f32 = jnp.float32
    xf = xf_ref[0]
    xt = xt_ref[0].reshape(RT, H)
    vxf = jnp.dot(xf, vew_ref[...], preferred_element_type=f32) + veb_ref[...]
    vx_tile = jnp.dot(xt, vew_ref[...], preferred_element_type=f32) + veb_ref[...]
    for r in range(RT):
        e_row = e_ref[0, r]
        ue = jnp.dot(e_row, uew_ref[...], preferred_element_type=f32) + ueb_ref[...]
        e_tmp = ue + vx_tile[r:r + 1] + vxf
        eo_ref[0, r] = e_row + jnp.maximum(e_tmp * ge_ref[...] + be_ref[...], 0.0)


def _sc_gather_body(table_hbm, idxg_hbm, idxe_hbm, outg_hbm, oute_hbm,
                    idx_v, idxe_v, rows0, rows1, rowse, sem0, sem1, seme):
    wid = lax.axis_index("s") * 2 + lax.axis_index("c")
    base = wid * ROWS_PER_W

    pltpu.sync_copy(idxg_hbm.at[wid], idx_v)
    pltpu.sync_copy(idxe_hbm.at[wid], idxe_v)
    pltpu.async_copy(table_hbm.at[idxe_v], rowse, seme)
    pltpu.async_copy(table_hbm.at[idx_v.at[0]], rows0, sem0)
    pltpu.async_copy(table_hbm.at[idx_v.at[1]], rows1, sem1)

    def step(s, carry):
        i0 = 2 * s
        i1 = i0 + 1
        pltpu.make_async_copy(table_hbm.at[idx_v.at[i0]], rows0, sem0).wait()
        pltpu.sync_copy(rows0, outg_hbm.at[pl.ds(base + i0 * CHUNK, CHUNK)])

        @pl.when(i0 + 2 < NCHUNK)
        def _():
            pltpu.async_copy(table_hbm.at[idx_v.at[i0 + 2]], rows0, sem0)

        pltpu.make_async_copy(table_hbm.at[idx_v.at[i1]], rows1, sem1).wait()
        pltpu.sync_copy(rows1, outg_hbm.at[pl.ds(base + i1 * CHUNK, CHUNK)])

        @pl.when(i1 + 2 < NCHUNK)
        def _():
            pltpu.async_copy(table_hbm.at[idx_v.at[i1 + 2]], rows1, sem1)

        return carry

    lax.fori_loop(0, NCHUNK // 2, step, 0)
    pltpu.make_async_copy(table_hbm.at[idxe_v], rowse, seme).wait()
    pltpu.sync_copy(rowse, oute_hbm.at[pl.ds(wid * E_PER_W, E_PER_W)])


_sc_gather = functools.partial(
    pl.kernel,
    out_type=[jax.ShapeDtypeStruct((G_ROWS, H), jnp.float32),
              jax.ShapeDtypeStruct((E_ROWS, H), jnp.float32)],
    mesh=plsc.VectorSubcoreMesh(core_axis_name="c", subcore_axis_name="s"),
    scratch_types=[
        pltpu.VMEM((NCHUNK, CHUNK), jnp.int32),
        pltpu.VMEM((E_PER_W,), jnp.int32),
        pltpu.VMEM((CHUNK, H), jnp.float32),
        pltpu.VMEM((CHUNK, H), jnp.float32),
        pltpu.VMEM((E_PER_W, H), jnp.float32),
        pltpu.SemaphoreType.DMA,
        pltpu.SemaphoreType.DMA,
        pltpu.SemaphoreType.DMA,
    ],
)(_sc_gather_body)


def _mlp_body(g1_ref, g2_ref, e1_ref, s1_ref, s2_ref, noise_ref,
              wa_ref, wb_ref, wc_ref, wd_ref, bp_ref,
              w1_ref, b1_ref, w2_ref, b2_ref, w3_ref, b3_ref,
              wo_ref, bo_ref,
              act_ref, pi_ref):
    f32 = jnp.float32
    a1 = jnp.dot(e1_ref[0], wa_ref[...], preferred_element_type=f32)
    a2 = jnp.dot(e1_ref[0], wb_ref[...], preferred_element_type=f32)
    h = (jnp.dot(s1_ref[...], a1, preferred_element_type=f32)
         + jnp.dot(s2_ref[...], a2, preferred_element_type=f32)
         + jnp.dot(g1_ref[0, 0], wc_ref[...], preferred_element_type=f32)
         + jnp.dot(g2_ref[0, 0], wd_ref[...], preferred_element_type=f32)
         + bp_ref[...])
    h = jnp.maximum(jnp.dot(h, w1_ref[...], preferred_element_type=f32) + b1_ref[...], 0.0)
    h = jnp.maximum(jnp.dot(h, w2_ref[...], preferred_element_type=f32) + b2_ref[...], 0.0)
    h = jnp.maximum(jnp.dot(h, w3_ref[...], preferred_element_type=f32) + b3_ref[...], 0.0)
    logits = jnp.dot(h, wo_ref[...], preferred_element_type=f32) + bo_ref[...]
    rowid = lax.broadcasted_iota(jnp.int32, (PP, 1), 0)
    logits = jnp.where(rowid < P, logits, f32(-1e30))
    z = logits + noise_ref[0]
    maxz = jnp.max(z)
    action = jnp.min(jnp.where(z >= maxz, rowid, jnp.int32(PP)))
    m = jnp.max(logits)
    lse = m + jnp.log(jnp.sum(jnp.exp(logits - m)))
    logit_a = jnp.sum(jnp.where(rowid == action, logits, 0.0))
    act_ref[0] = action[None, None]
    pi_ref[0] = (logit_a - lse)[None, None]


def _full(shape):
    nd = len(shape)
    return pl.BlockSpec(shape, lambda *a: (0,) * nd)


def kernel(x_edges, x_edges_values, x_nodes_coord, x_tour, x_best_tour,
           x_tour_directed, params):
    p = params
    f32 = jnp.float32
    cbn = np.float32(1.0 / np.sqrt(1.0 + 1e-5))
    xt = x_tour.astype(jnp.int32)
    xb = x_best_tour.astype(jnp.int32)

    vals4 = x_edges_values.reshape(B, V, V, 1)
    t4 = xt.reshape(B, V, V, 1)
    b4 = xb.reshape(B, V, V, 1)
    wev = p['W_evals'].reshape(1, HH)

    node_w_specs = [_full((H, H)), _full((1, H)),
                    _full((H, H)), _full((1, H)),
                    _full((H, H)), _full((1, H))]
    bvh_spec = pl.BlockSpec((1, V, H), lambda b: (b, 0, 0))

    par2 = pltpu.CompilerParams(dimension_semantics=("parallel", "parallel"))
    coords4 = x_nodes_coord.reshape(B, V, 1, 2)
    row_spec = pl.BlockSpec((1, RT, 1, H), lambda b, i: (b, i, 0, 0))
    w_spec = pl.BlockSpec((H, H), lambda b, i: (0, 0))
    h_spec = pl.BlockSpec((1, H), lambda b, i: (0, 0))
    xf_spec = pl.BlockSpec((1, V, H), lambda b, i: (b, 0, 0))
    e_spec = pl.BlockSpec((1, RT, V, H), lambda b, i: (b, i, 0, 0))
    hh_spec = pl.BlockSpec((1, HH), lambda b, i: (0, 0))
    emb_spec = pl.BlockSpec((3, HH), lambda b, i: (0, 0))

    e = None
    x4 = None
    for li, lp in enumerate(p['layers']):
        last = li == len(p['layers']) - 1
        ge = (lp['bn_e'][0] * cbn).reshape(1, H)
        be = lp['bn_e'][1].reshape(1, H)
        if last:
            e = pl.pallas_call(
                _edge_last_body,
                grid=(B, V // RT),
                in_specs=[e_spec, xf_spec, row_spec,
                          w_spec, h_spec, w_spec, h_spec, h_spec, h_spec],
                out_specs=e_spec,
                out_shape=jax.ShapeDtypeStruct((B, V, V, H), f32),
                compiler_params=par2,
            )(e, x4.reshape(B, V, H), x4,
              lp['Ve'][0], lp['Ve'][1].reshape(1, H),
              lp['Ue'][0], lp['Ue'][1].reshape(1, H), ge, be)
            break

        gn = (lp['bn_n'][0] * cbn).reshape(1, H)
        bn = lp['bn_n'][1].reshape(1, H)
        node_w_args = (lp['Ve'][0], lp['Ve'][1].reshape(1, H),
                       lp['Un'][0], lp['Un'][1].reshape(1, H),
                       lp['Vn'][0], lp['Vn'][1].reshape(1, H))
        node_w_sp = [w_spec, h_spec, w_spec, h_spec, w_spec, h_spec]
        tail_sp = node_w_sp + [w_spec, h_spec, h_spec, h_spec, h_spec, h_spec]
        tail_args = node_w_args + (lp['Ue'][0], lp['Ue'][1].reshape(1, H),
                                   ge, be, gn, bn)
        out_specs = [e_spec, row_spec]
        out_shape = [jax.ShapeDtypeStruct((B, V, V, H), f32),
                     jax.ShapeDtypeStruct((B, V, 1, H), f32)]
        if li == 0:
            e, x4 = pl.pallas_call(
                _edge1_body,
                grid=(B, V // RT),
                in_specs=[
                    pl.BlockSpec((1, RT, V, 1), lambda b, i: (b, i, 0, 0)),
                    pl.BlockSpec((1, RT, V, 1), lambda b, i: (b, i, 0, 0)),
                    pl.BlockSpec((1, RT, V, 1), lambda b, i: (b, i, 0, 0)),
                    hh_spec, emb_spec, emb_spec,
                    pl.BlockSpec((1, V, 2), lambda b, i: (b, 0, 0)),
                    pl.BlockSpec((1, RT, 1, 2), lambda b, i: (b, i, 0, 0)),
                    pl.BlockSpec((2, H), lambda b, i: (0, 0)),
                ] + tail_sp,
                out_specs=out_specs,
                out_shape=out_shape,
                compiler_params=par2,
            )(vals4, t4, b4, wev, p['emb0'], p['emb1'],
              x_nodes_coord, coords4, p['W_nodes'], *tail_args)
        else:
            e, x4 = pl.pallas_call(
                _edge_body,
                grid=(B, V // RT),
                in_specs=[e_spec, xf_spec, row_spec] + tail_sp,
                out_specs=out_specs,
                out_shape=out_shape,
                compiler_params=par2,
            )(e, x4.reshape(B, V, H), x4, *tail_args)

    first = jnp.argmax(xt, axis=2).astype(jnp.int32)
    last = (V - 1) - jnp.argmax(xt[:, :, ::-1], axis=2).astype(jnp.int32)
    ii = jnp.arange(V, dtype=jnp.int32)[None, :]
    cnt = (first > ii).astype(jnp.int32) + (last > ii).astype(jnp.int32)
    start = jnp.cumsum(cnt, axis=1) - cnt
    kk = jnp.arange(V, dtype=jnp.int32)
    i_e = jnp.sum((start[:, :, None] <= kk[None, None, :]).astype(jnp.int32),
                  axis=1) - 1
    f_i = jnp.take_along_axis(first, i_e, axis=1)
    l_i = jnp.take_along_axis(last, i_e, axis=1)
    s_i = jnp.take_along_axis(start, i_e, axis=1)
    firstj = jnp.where(f_i > i_e, f_i, l_i)
    j_e = jnp.where(kk[None, :] == s_i, firstj, l_i)

    d = jnp.take_along_axis(x_tour_directed.reshape(B, V * V),
                            i_e * V + j_e, axis=1)
    U = jnp.where(d, i_e, j_e)
    Vv = jnp.where(d, j_e, i_e)

    boff = (jnp.arange(B, dtype=jnp.int32) * (V * V))[:, None]
    Uk1, Uk2 = U[:, RS_PAD], U[:, CS_PAD]
    Vk1, Vk2 = Vv[:, RS_PAD], Vv[:, CS_PAD]
    idx_g = jnp.stack([
        boff + Uk1 * V + Uk2,
        boff + Vk1 * V + Vk2,
    ]).reshape(NW, NCHUNK, CHUNK)
    idx_e = jnp.pad((boff + U * V + Vv).reshape(B * V),
                    (0, E_ROWS - B * V)).reshape(NW, E_PER_W)

    table = e.reshape(B * V * V, H)
    rows_g, rows_e = _sc_gather(table, idx_g, idx_e)
    quad = rows_g.reshape(2, B, PP, H)
    e1 = rows_e[:B * V].reshape(B, V, H)

    noise = jax.random.gumbel(jax.random.key(42), (B, P), f32)
    noise = jnp.pad(noise, ((0, 0), (0, PP - P))).reshape(B, PP, 1)
    Wp, bp = p['pre_act']
    w1, b1 = p['act_hidden'][0]
    w2, b2 = p['act_hidden'][1]
    w3, b3 = p['act_hidden'][2]
    wo, bo = p['act_out']
    tab_spec = lambda t: pl.BlockSpec((1, 1, PP, H), lambda b, _t=t: (_t, b, 0, 0))
    act2, pi2 = pl.pallas_call(
        _mlp_body,
        grid=(B,),
        in_specs=[
            tab_spec(0), tab_spec(1),
            pl.BlockSpec((1, V, H), lambda b: (b, 0, 0)),
            _full((PP, V)), _full((PP, V)),
            pl.BlockSpec((1, PP, 1), lambda b: (b, 0, 0)),
            _full((H, H)), _full((H, H)), _full((H, H)), _full((H, H)),
            _full((1, H)),
            _full((H, H)), _full((1, H)),
            _full((H, H)), _full((1, H)),
            _full((H, H)), _full((1, H)),
            _full((H, 1)), _full((1, 1)),
        ],
        out_specs=[pl.BlockSpec((1, 1, 1), lambda b: (b, 0, 0)),
                   pl.BlockSpec((1, 1, 1), lambda b: (b, 0, 0))],
        out_shape=[jax.ShapeDtypeStruct((B, 1, 1), jnp.int32),
                   jax.ShapeDtypeStruct((B, 1, 1), f32)],
    )(quad, quad, e1, jnp.asarray(S1_ONEHOT), jnp.asarray(S2_ONEHOT), noise,
      Wp[0:H], Wp[H:2 * H], Wp[2 * H:3 * H], Wp[3 * H:4 * H], bp.reshape(1, H),
      w1, b1.reshape(1, H), w2, b2.reshape(1, H), w3, b3.reshape(1, H),
      wo, bo.reshape(1, 1))

    actions = act2[:, 0, 0]
    pi = pi2[:, 0, 0]

    k1 = jnp.asarray(RS_PAD)[actions]
    k2 = jnp.asarray(CS_PAD)[actions]
    barange = jnp.arange(B, dtype=jnp.int32)

    def edge_row(kidx):
        return jnp.stack([
            barange,
            jnp.take_along_axis(i_e, kidx[:, None], axis=1)[:, 0],
            jnp.take_along_axis(j_e, kidx[:, None], axis=1)[:, 0],
        ], axis=1)

    edges = jnp.stack([edge_row(k1), edge_row(k2)], axis=1)
    return edges, pi, actions


# speedup vs baseline: 7.3128x; 1.0209x over previous
import functools

import numpy as np
import jax
import jax.numpy as jnp
from jax import lax
from jax.experimental import pallas as pl
from jax.experimental.pallas import tpu as pltpu
from jax.experimental.pallas import tpu_sc as plsc

B, V, H = 8, 100, 128
HH = H // 2
P = V * (V - 1) // 2
PP = 4992
G_ROWS = 2 * B * PP
NW = 32
CHUNK = 96
ROWS_PER_W = G_ROWS // NW
NCHUNK = ROWS_PER_W // CHUNK
E_ROWS = 1024
E_PER_W = E_ROWS // NW
RT = 25

_RS, _CS = np.triu_indices(V, 1)
RS_PAD = np.concatenate([_RS, np.zeros(PP - P, np.int64)]).astype(np.int32)
CS_PAD = np.concatenate([_CS, np.zeros(PP - P, np.int64)]).astype(np.int32)
S1_ONEHOT = np.zeros((PP, V), np.float32)
S1_ONEHOT[np.arange(PP), RS_PAD] = 1.0
S2_ONEHOT = np.zeros((PP, V), np.float32)
S2_ONEHOT[np.arange(PP), CS_PAD] = 1.0


def _edge_update(e_row, r, vxf, vx_tile, ux_tile, vnf, x_tile,
                 uew_ref, ueb_ref, ge_ref, be_ref, gn_ref, bn_ref,
                 eo_ref, xo_ref):
    ue = jnp.dot(e_row, uew_ref[...], preferred_element_type=jnp.float32) + ueb_ref[...]
    e_tmp = ue + vx_tile[r:r + 1] + vxf
    gate = 1.0 / (1.0 + jnp.exp(-e_tmp))
    num = jnp.sum(gate * vnf, axis=0, keepdims=True)
    den = 1e-20 + jnp.sum(gate, axis=0, keepdims=True)
    x_tmp = ux_tile[r:r + 1] + num / den
    eo_ref[0, r] = e_row + jnp.maximum(e_tmp * ge_ref[...] + be_ref[...], 0.0)
    xo_ref[0, r] = x_tile[r:r + 1] + jnp.maximum(x_tmp * gn_ref[...] + bn_ref[...], 0.0)


def _node_products(xf, xt, vew_ref, veb_ref, unw_ref, unb_ref,
                   vnw_ref, vnb_ref):
    f32 = jnp.float32
    vxf = jnp.dot(xf, vew_ref[...], preferred_element_type=f32) + veb_ref[...]
    vnf = jnp.dot(xf, vnw_ref[...], preferred_element_type=f32) + vnb_ref[...]
    vx_tile = jnp.dot(xt, vew_ref[...], preferred_element_type=f32) + veb_ref[...]
    ux_tile = jnp.dot(xt, unw_ref[...], preferred_element_type=f32) + unb_ref[...]
    return vxf, vnf, vx_tile, ux_tile


def _edge1_body(vals_ref, tour_ref, best_ref, wev_ref, emb0_ref, emb1_ref,
                coordf_ref, coordt_ref, wn_ref,
                vew_ref, veb_ref, unw_ref, unb_ref, vnw_ref, vnb_ref,
                uew_ref, ueb_ref, ge_ref, be_ref, gn_ref, bn_ref,
                eo_ref, xo_ref):
    cf = coordf_ref[0]
    xf = cf[:, 0:1] * wn_ref[0:1, :] + cf[:, 1:2] * wn_ref[1:2, :]
    ct = coordt_ref[0].reshape(RT, 2)
    xt = ct[:, 0:1] * wn_ref[0:1, :] + ct[:, 1:2] * wn_ref[1:2, :]
    vxf, vnf, vx_tile, ux_tile = _node_products(
        xf, xt, vew_ref, veb_ref, unw_ref, unb_ref, vnw_ref, vnb_ref)
    for r in range(RT):
        ev = vals_ref[0, r] * wev_ref[...]
        tags = (jnp.where(tour_ref[0, r] > 0, emb0_ref[1:2, :], emb0_ref[0:1, :])
                + jnp.where(best_ref[0, r] > 0, emb1_ref[1:2, :], emb1_ref[0:1, :]))
        e_row = jnp.concatenate([ev, tags], axis=-1)
        _edge_update(e_row, r, vxf, vx_tile, ux_tile, vnf, xt,
                     uew_ref, ueb_ref, ge_ref, be_ref, gn_ref, bn_ref,
                     eo_ref, xo_ref)


def _edge_body(e_ref, xf_ref, xt_ref,
               vew_ref, veb_ref, unw_ref, unb_ref, vnw_ref, vnb_ref,
               uew_ref, ueb_ref, ge_ref, be_ref, gn_ref, bn_ref,
               eo_ref, xo_ref):
    xf = xf_ref[0]
    xt = xt_ref[0].reshape(RT, H)
    vxf, vnf, vx_tile, ux_tile = _node_products(
        xf, xt, vew_ref, veb_ref, unw_ref, unb_ref, vnw_ref, vnb_ref)
    for r in range(RT):
        _edge_update(e_ref[0, r], r, vxf, vx_tile, ux_tile, vnf, xt,
                     uew_ref, ueb_ref, ge_ref, be_ref, gn_ref, bn_ref,
                     eo_ref, xo_ref)


def _edge_last_body(e_ref, xf_ref, xt_ref, vew_ref, veb_ref,
                    uew_ref, ueb_ref, ge_ref, be_ref, eo_ref):
    f32 = jnp.float32
    xf = xf_ref[0]
    xt = xt_ref[0].reshape(RT, H)
    vxf = jnp.dot(xf, vew_ref[...], preferred_element_type=f32) + veb_ref[...]
    vx_tile = jnp.dot(xt, vew_ref[...], preferred_element_type=f32) + veb_ref[...]
    for r in range(RT):
        e_row = e_ref[0, r]
        ue = jnp.dot(e_row, uew_ref[...], preferred_element_type=f32) + ueb_ref[...]
        e_tmp = ue + vx_tile[r:r + 1] + vxf
        eo_ref[0, r] = e_row + jnp.maximum(e_tmp * ge_ref[...] + be_ref[...], 0.0)


def _sc_gather_body(table_hbm, idxg_hbm, idxe_hbm, outg_hbm, oute_hbm,
                    idx_v, idxe_v, rows0, rows1, rowse, sem0, sem1, seme):
    wid = lax.axis_index("s") * 2 + lax.axis_index("c")
    base = wid * ROWS_PER_W

    pltpu.sync_copy(idxg_hbm.at[wid], idx_v)
    pltpu.sync_copy(idxe_hbm.at[wid], idxe_v)
    pltpu.async_copy(table_hbm.at[idxe_v], rowse, seme)
    pltpu.async_copy(table_hbm.at[idx_v.at[0]], rows0, sem0)
    pltpu.async_copy(table_hbm.at[idx_v.at[1]], rows1, sem1)

    def step(s, carry):
        i0 = 2 * s
        i1 = i0 + 1
        pltpu.make_async_copy(table_hbm.at[idx_v.at[i0]], rows0, sem0).wait()
        pltpu.sync_copy(rows0, outg_hbm.at[pl.ds(base + i0 * CHUNK, CHUNK)])

        @pl.when(i0 + 2 < NCHUNK)
        def _():
            pltpu.async_copy(table_hbm.at[idx_v.at[i0 + 2]], rows0, sem0)

        pltpu.make_async_copy(table_hbm.at[idx_v.at[i1]], rows1, sem1).wait()
        pltpu.sync_copy(rows1, outg_hbm.at[pl.ds(base + i1 * CHUNK, CHUNK)])

        @pl.when(i1 + 2 < NCHUNK)
        def _():
            pltpu.async_copy(table_hbm.at[idx_v.at[i1 + 2]], rows1, sem1)

        return carry

    lax.fori_loop(0, NCHUNK // 2, step, 0)
    pltpu.make_async_copy(table_hbm.at[idxe_v], rowse, seme).wait()
    pltpu.sync_copy(rowse, oute_hbm.at[pl.ds(wid * E_PER_W, E_PER_W)])


_sc_gather = functools.partial(
    pl.kernel,
    out_type=[jax.ShapeDtypeStruct((G_ROWS, H), jnp.float32),
              jax.ShapeDtypeStruct((E_ROWS, H), jnp.float32)],
    mesh=plsc.VectorSubcoreMesh(core_axis_name="c", subcore_axis_name="s"),
    scratch_types=[
        pltpu.VMEM((NCHUNK, CHUNK), jnp.int32),
        pltpu.VMEM((E_PER_W,), jnp.int32),
        pltpu.VMEM((CHUNK, H), jnp.float32),
        pltpu.VMEM((CHUNK, H), jnp.float32),
        pltpu.VMEM((E_PER_W, H), jnp.float32),
        pltpu.SemaphoreType.DMA,
        pltpu.SemaphoreType.DMA,
        pltpu.SemaphoreType.DMA,
    ],
)(_sc_gather_body)


def _mlp_body(g1_ref, g2_ref, e1_ref, s1_ref, s2_ref, noise_ref,
              wa_ref, wb_ref, wc_ref, wd_ref, bp_ref,
              w1_ref, b1_ref, w2_ref, b2_ref, w3_ref, b3_ref,
              wo_ref, bo_ref,
              act_ref, pi_ref):
    f32 = jnp.float32
    a1 = jnp.dot(e1_ref[0], wa_ref[...], preferred_element_type=f32)
    a2 = jnp.dot(e1_ref[0], wb_ref[...], preferred_element_type=f32)
    h = (jnp.dot(s1_ref[...], a1, preferred_element_type=f32)
         + jnp.dot(s2_ref[...], a2, preferred_element_type=f32)
         + jnp.dot(g1_ref[0, 0], wc_ref[...], preferred_element_type=f32)
         + jnp.dot(g2_ref[0, 0], wd_ref[...], preferred_element_type=f32)
         + bp_ref[...])
    h = jnp.maximum(jnp.dot(h, w1_ref[...], preferred_element_type=f32) + b1_ref[...], 0.0)
    h = jnp.maximum(jnp.dot(h, w2_ref[...], preferred_element_type=f32) + b2_ref[...], 0.0)
    h = jnp.maximum(jnp.dot(h, w3_ref[...], preferred_element_type=f32) + b3_ref[...], 0.0)
    logits = jnp.dot(h, wo_ref[...], preferred_element_type=f32) + bo_ref[...]
    rowid = lax.broadcasted_iota(jnp.int32, (PP, 1), 0)
    logits = jnp.where(rowid < P, logits, f32(-1e30))
    z = logits + noise_ref[0]
    maxz = jnp.max(z)
    action = jnp.min(jnp.where(z >= maxz, rowid, jnp.int32(PP)))
    m = jnp.max(logits)
    lse = m + jnp.log(jnp.sum(jnp.exp(logits - m)))
    logit_a = jnp.sum(jnp.where(rowid == action, logits, 0.0))
    act_ref[0] = action[None, None]
    pi_ref[0] = (logit_a - lse)[None, None]


def _full(shape):
    nd = len(shape)
    return pl.BlockSpec(shape, lambda *a: (0,) * nd)


def kernel(x_edges, x_edges_values, x_nodes_coord, x_tour, x_best_tour,
           x_tour_directed, params):
    p = params
    f32 = jnp.float32
    cbn = np.float32(1.0 / np.sqrt(1.0 + 1e-5))
    xt = x_tour.astype(jnp.int32)
    xb = x_best_tour.astype(jnp.int32)

    vals4 = x_edges_values.reshape(B, V, V, 1)
    t4 = xt.reshape(B, V, V, 1)
    b4 = xb.reshape(B, V, V, 1)
    wev = p['W_evals'].reshape(1, HH)

    node_w_specs = [_full((H, H)), _full((1, H)),
                    _full((H, H)), _full((1, H)),
                    _full((H, H)), _full((1, H))]
    bvh_spec = pl.BlockSpec((1, V, H), lambda b: (b, 0, 0))

    par2 = pltpu.CompilerParams(dimension_semantics=("parallel", "parallel"))
    coords4 = x_nodes_coord.reshape(B, V, 1, 2)
    row_spec = pl.BlockSpec((1, RT, 1, H), lambda b, i: (b, i, 0, 0))
    w_spec = pl.BlockSpec((H, H), lambda b, i: (0, 0))
    h_spec = pl.BlockSpec((1, H), lambda b, i: (0, 0))
    xf_spec = pl.BlockSpec((1, V, H), lambda b, i: (b, 0, 0))
    e_spec = pl.BlockSpec((1, RT, V, H), lambda b, i: (b, i, 0, 0))
    hh_spec = pl.BlockSpec((1, HH), lambda b, i: (0, 0))
    emb_spec = pl.BlockSpec((3, HH), lambda b, i: (0, 0))

    e = None
    x4 = None
    for li, lp in enumerate(p['layers']):
        last = li == len(p['layers']) - 1
        ge = (lp['bn_e'][0] * cbn).reshape(1, H)
        be = lp['bn_e'][1].reshape(1, H)
        if last:
            e = pl.pallas_call(
                _edge_last_body,
                grid=(B, V // RT),
                in_specs=[e_spec, xf_spec, row_spec,
                          w_spec, h_spec, w_spec, h_spec, h_spec, h_spec],
                out_specs=e_spec,
                out_shape=jax.ShapeDtypeStruct((B, V, V, H), f32),
                compiler_params=par2,
            )(e, x4.reshape(B, V, H), x4,
              lp['Ve'][0], lp['Ve'][1].reshape(1, H),
              lp['Ue'][0], lp['Ue'][1].reshape(1, H), ge, be)
            break

        gn = (lp['bn_n'][0] * cbn).reshape(1, H)
        bn = lp['bn_n'][1].reshape(1, H)
        node_w_args = (lp['Ve'][0], lp['Ve'][1].reshape(1, H),
                       lp['Un'][0], lp['Un'][1].reshape(1, H),
                       lp['Vn'][0], lp['Vn'][1].reshape(1, H))
        node_w_sp = [w_spec, h_spec, w_spec, h_spec, w_spec, h_spec]
        tail_sp = node_w_sp + [w_spec, h_spec, h_spec, h_spec, h_spec, h_spec]
        tail_args = node_w_args + (lp['Ue'][0], lp['Ue'][1].reshape(1, H),
                                   ge, be, gn, bn)
        out_specs = [e_spec, row_spec]
        out_shape = [jax.ShapeDtypeStruct((B, V, V, H), f32),
                     jax.ShapeDtypeStruct((B, V, 1, H), f32)]
        if li == 0:
            e, x4 = pl.pallas_call(
                _edge1_body,
                grid=(B, V // RT),
                in_specs=[
                    pl.BlockSpec((1, RT, V, 1), lambda b, i: (b, i, 0, 0)),
                    pl.BlockSpec((1, RT, V, 1), lambda b, i: (b, i, 0, 0)),
                    pl.BlockSpec((1, RT, V, 1), lambda b, i: (b, i, 0, 0)),
                    hh_spec, emb_spec, emb_spec,
                    pl.BlockSpec((1, V, 2), lambda b, i: (b, 0, 0)),
                    pl.BlockSpec((1, RT, 1, 2), lambda b, i: (b, i, 0, 0)),
                    pl.BlockSpec((2, H), lambda b, i: (0, 0)),
                ] + tail_sp,
                out_specs=out_specs,
                out_shape=out_shape,
                compiler_params=par2,
            )(vals4, t4, b4, wev, p['emb0'], p['emb1'],
              x_nodes_coord, coords4, p['W_nodes'], *tail_args)
        else:
            e, x4 = pl.pallas_call(
                _edge_body,
                grid=(B, V // RT),
                in_specs=[e_spec, xf_spec, row_spec] + tail_sp,
                out_specs=out_specs,
                out_shape=out_shape,
                compiler_params=par2,
            )(e, x4.reshape(B, V, H), x4, *tail_args)

    first = jnp.argmax(xt, axis=2).astype(jnp.int32)
    last = (V - 1) - jnp.argmax(xt[:, :, ::-1], axis=2).astype(jnp.int32)
    ii = jnp.arange(V, dtype=jnp.int32)[None, :]
    cnt = (first > ii).astype(jnp.int32) + (last > ii).astype(jnp.int32)
    start = jnp.cumsum(cnt, axis=1) - cnt
    kk = jnp.arange(V, dtype=jnp.int32)
    i_e = jnp.sum((start[:, :, None] <= kk[None, None, :]).astype(jnp.int32),
                  axis=1) - 1
    f_i = jnp.take_along_axis(first, i_e, axis=1)
    l_i = jnp.take_along_axis(last, i_e, axis=1)
    s_i = jnp.take_along_axis(start, i_e, axis=1)
    firstj = jnp.where(f_i > i_e, f_i, l_i)
    j_e = jnp.where(kk[None, :] == s_i, firstj, l_i)

    d = jnp.take_along_axis(x_tour_directed.reshape(B, V * V),
                            i_e * V + j_e, axis=1)
    U = jnp.where(d, i_e, j_e)
    Vv = jnp.where(d, j_e, i_e)

    boff = (jnp.arange(B, dtype=jnp.int32) * (V * V))[:, None]
    Uk1, Uk2 = U[:, RS_PAD], U[:, CS_PAD]
    Vk1, Vk2 = Vv[:, RS_PAD], Vv[:, CS_PAD]
    idx_g = jnp.stack([
        boff + Uk1 * V + Uk2,
        boff + Vk1 * V + Vk2,
    ]).reshape(NW, NCHUNK, CHUNK)
    idx_e = jnp.pad((boff + U * V + Vv).reshape(B * V),
                    (0, E_ROWS - B * V)).reshape(NW, E_PER_W)

    table = e.reshape(B * V * V, H)
    rows_g, rows_e = _sc_gather(table, idx_g, idx_e)
    quad = rows_g.reshape(2, B, PP, H)
    e1 = rows_e[:B * V].reshape(B, V, H)

    noise = jax.random.gumbel(jax.random.key(42), (B, P), f32)
    noise = jnp.pad(noise, ((0, 0), (0, PP - P))).reshape(B, PP, 1)
    Wp, bp = p['pre_act']
    w1, b1 = p['act_hidden'][0]
    w2, b2 = p['act_hidden'][1]
    w3, b3 = p['act_hidden'][2]
    wo, bo = p['act_out']
    tab_spec = lambda t: pl.BlockSpec((1, 1, PP, H), lambda b, _t=t: (_t, b, 0, 0))
    act2, pi2 = pl.pallas_call(
        _mlp_body,
        grid=(B,),
        in_specs=[
            tab_spec(0), tab_spec(1),
            pl.BlockSpec((1, V, H), lambda b: (b, 0, 0)),
            _full((PP, V)), _full((PP, V)),
            pl.BlockSpec((1, PP, 1), lambda b: (b, 0, 0)),
            _full((H, H)), _full((H, H)), _full((H, H)), _full((H, H)),
            _full((1, H)),
            _full((H, H)), _full((1, H)),
            _full((H, H)), _full((1, H)),
            _full((H, H)), _full((1, H)),
            _full((H, 1)), _full((1, 1)),
        ],
        out_specs=[pl.BlockSpec((1, 1, 1), lambda b: (b, 0, 0)),
                   pl.BlockSpec((1, 1, 1), lambda b: (b, 0, 0))],
        out_shape=[jax.ShapeDtypeStruct((B, 1, 1), jnp.int32),
                   jax.ShapeDtypeStruct((B, 1, 1), f32)],
    )(quad, quad, e1, jnp.asarray(S1_ONEHOT), jnp.asarray(S2_ONEHOT), noise,
      Wp[0:H], Wp[H:2 * H], Wp[2 * H:3 * H], Wp[3 * H:4 * H], bp.reshape(1, H),
      w1, b1.reshape(1, H), w2, b2.reshape(1, H), w3, b3.reshape(1, H),
      wo, bo.reshape(1, 1))

    actions = act2[:, 0, 0]
    pi = pi2[:, 0, 0]

    k1 = jnp.asarray(RS_PAD)[actions]
    k2 = jnp.asarray(CS_PAD)[actions]
    barange = jnp.arange(B, dtype=jnp.int32)

    def edge_row(kidx):
        return jnp.stack([
            barange,
            jnp.take_along_axis(i_e, kidx[:, None], axis=1)[:, 0],
            jnp.take_along_axis(j_e, kidx[:, None], axis=1)[:, 0],
        ], axis=1)

    edges = jnp.stack([edge_row(k1), edge_row(k2)], axis=1)
    return edges, pi, actions


# speedup vs baseline: 7.5887x; 1.0377x over previous
import functools

import numpy as np
import jax
import jax.numpy as jnp
from jax import lax
from jax.experimental import pallas as pl
from jax.experimental.pallas import tpu as pltpu
from jax.experimental.pallas import tpu_sc as plsc

B, V, H = 8, 100, 128
HH = H // 2
P = V * (V - 1) // 2
PP = 4992
G_ROWS = 2 * B * PP
NW = 32
CHUNK = 96
ROWS_PER_W = G_ROWS // NW
NCHUNK = ROWS_PER_W // CHUNK
E_ROWS = 1024
E_PER_W = E_ROWS // NW
RT = 50

_RS, _CS = np.triu_indices(V, 1)
RS_PAD = np.concatenate([_RS, np.zeros(PP - P, np.int64)]).astype(np.int32)
CS_PAD = np.concatenate([_CS, np.zeros(PP - P, np.int64)]).astype(np.int32)
S1_ONEHOT = np.zeros((PP, V), np.float32)
S1_ONEHOT[np.arange(PP), RS_PAD] = 1.0
S2_ONEHOT = np.zeros((PP, V), np.float32)
S2_ONEHOT[np.arange(PP), CS_PAD] = 1.0


def _edge_update(e_row, r, vxf, vx_tile, ux_tile, vnf, x_tile,
                 uew_ref, ueb_ref, ge_ref, be_ref, gn_ref, bn_ref,
                 eo_ref, xo_ref):
    ue = jnp.dot(e_row, uew_ref[...], preferred_element_type=jnp.float32) + ueb_ref[...]
    e_tmp = ue + vx_tile[r:r + 1] + vxf
    gate = 1.0 / (1.0 + jnp.exp(-e_tmp))
    num = jnp.sum(gate * vnf, axis=0, keepdims=True)
    den = 1e-20 + jnp.sum(gate, axis=0, keepdims=True)
    x_tmp = ux_tile[r:r + 1] + num / den
    eo_ref[0, r] = e_row + jnp.maximum(e_tmp * ge_ref[...] + be_ref[...], 0.0)
    xo_ref[0, r] = x_tile[r:r + 1] + jnp.maximum(x_tmp * gn_ref[...] + bn_ref[...], 0.0)


def _node_products(xf, xt, vew_ref, veb_ref, unw_ref, unb_ref,
                   vnw_ref, vnb_ref):
    f32 = jnp.float32
    vxf = jnp.dot(xf, vew_ref[...], preferred_element_type=f32) + veb_ref[...]
    vnf = jnp.dot(xf, vnw_ref[...], preferred_element_type=f32) + vnb_ref[...]
    vx_tile = jnp.dot(xt, vew_ref[...], preferred_element_type=f32) + veb_ref[...]
    ux_tile = jnp.dot(xt, unw_ref[...], preferred_element_type=f32) + unb_ref[...]
    return vxf, vnf, vx_tile, ux_tile


def _edge1_body(vals_ref, tour_ref, best_ref, wev_ref, emb0_ref, emb1_ref,
                coordf_ref, coordt_ref, wn_ref,
                vew_ref, veb_ref, unw_ref, unb_ref, vnw_ref, vnb_ref,
                uew_ref, ueb_ref, ge_ref, be_ref, gn_ref, bn_ref,
                eo_ref, xo_ref):
    cf = coordf_ref[0]
    xf = cf[:, 0:1] * wn_ref[0:1, :] + cf[:, 1:2] * wn_ref[1:2, :]
    ct = coordt_ref[0].reshape(RT, 2)
    xt = ct[:, 0:1] * wn_ref[0:1, :] + ct[:, 1:2] * wn_ref[1:2, :]
    vxf, vnf, vx_tile, ux_tile = _node_products(
        xf, xt, vew_ref, veb_ref, unw_ref, unb_ref, vnw_ref, vnb_ref)
    for r in range(RT):
        ev = vals_ref[0, r] * wev_ref[...]
        tags = (jnp.where(tour_ref[0, r] > 0, emb0_ref[1:2, :], emb0_ref[0:1, :])
                + jnp.where(best_ref[0, r] > 0, emb1_ref[1:2, :], emb1_ref[0:1, :]))
        e_row = jnp.concatenate([ev, tags], axis=-1)
        _edge_update(e_row, r, vxf, vx_tile, ux_tile, vnf, xt,
                     uew_ref, ueb_ref, ge_ref, be_ref, gn_ref, bn_ref,
                     eo_ref, xo_ref)


def _edge_body(e_ref, xf_ref, xt_ref,
               vew_ref, veb_ref, unw_ref, unb_ref, vnw_ref, vnb_ref,
               uew_ref, ueb_ref, ge_ref, be_ref, gn_ref, bn_ref,
               eo_ref, xo_ref):
    xf = xf_ref[0]
    xt = xt_ref[0].reshape(RT, H)
    vxf, vnf, vx_tile, ux_tile = _node_products(
        xf, xt, vew_ref, veb_ref, unw_ref, unb_ref, vnw_ref, vnb_ref)
    for r in range(RT):
        _edge_update(e_ref[0, r], r, vxf, vx_tile, ux_tile, vnf, xt,
                     uew_ref, ueb_ref, ge_ref, be_ref, gn_ref, bn_ref,
                     eo_ref, xo_ref)


def _edge_last_body(e_ref, xf_ref, xt_ref, vew_ref, veb_ref,
                    uew_ref, ueb_ref, ge_ref, be_ref, eo_ref):
    f32 = jnp.float32
    xf = xf_ref[0]
    xt = xt_ref[0].reshape(RT, H)
    vxf = jnp.dot(xf, vew_ref[...], preferred_element_type=f32) + veb_ref[...]
    vx_tile = jnp.dot(xt, vew_ref[...], preferred_element_type=f32) + veb_ref[...]
    for r in range(RT):
        e_row = e_ref[0, r]
        ue = jnp.dot(e_row, uew_ref[...], preferred_element_type=f32) + ueb_ref[...]
        e_tmp = ue + vx_tile[r:r + 1] + vxf
        eo_ref[0, r] = e_row + jnp.maximum(e_tmp * ge_ref[...] + be_ref[...], 0.0)


def _sc_gather_body(table_hbm, idxg_hbm, idxe_hbm, outg_hbm, oute_hbm,
                    idx_v, idxe_v, rows0, rows1, rowse, sem0, sem1, seme):
    wid = lax.axis_index("s") * 2 + lax.axis_index("c")
    base = wid * ROWS_PER_W

    pltpu.sync_copy(idxg_hbm.at[wid], idx_v)
    pltpu.sync_copy(idxe_hbm.at[wid], idxe_v)
    pltpu.async_copy(table_hbm.at[idxe_v], rowse, seme)
    pltpu.async_copy(table_hbm.at[idx_v.at[0]], rows0, sem0)
    pltpu.async_copy(table_hbm.at[idx_v.at[1]], rows1, sem1)

    def step(s, carry):
        i0 = 2 * s
        i1 = i0 + 1
        pltpu.make_async_copy(table_hbm.at[idx_v.at[i0]], rows0, sem0).wait()
        pltpu.sync_copy(rows0, outg_hbm.at[pl.ds(base + i0 * CHUNK, CHUNK)])

        @pl.when(i0 + 2 < NCHUNK)
        def _():
            pltpu.async_copy(table_hbm.at[idx_v.at[i0 + 2]], rows0, sem0)

        pltpu.make_async_copy(table_hbm.at[idx_v.at[i1]], rows1, sem1).wait()
        pltpu.sync_copy(rows1, outg_hbm.at[pl.ds(base + i1 * CHUNK, CHUNK)])

        @pl.when(i1 + 2 < NCHUNK)
        def _():
            pltpu.async_copy(table_hbm.at[idx_v.at[i1 + 2]], rows1, sem1)

        return carry

    lax.fori_loop(0, NCHUNK // 2, step, 0)
    pltpu.make_async_copy(table_hbm.at[idxe_v], rowse, seme).wait()
    pltpu.sync_copy(rowse, oute_hbm.at[pl.ds(wid * E_PER_W, E_PER_W)])


_sc_gather = functools.partial(
    pl.kernel,
    out_type=[jax.ShapeDtypeStruct((G_ROWS, H), jnp.float32),
              jax.ShapeDtypeStruct((E_ROWS, H), jnp.float32)],
    mesh=plsc.VectorSubcoreMesh(core_axis_name="c", subcore_axis_name="s"),
    scratch_types=[
        pltpu.VMEM((NCHUNK, CHUNK), jnp.int32),
        pltpu.VMEM((E_PER_W,), jnp.int32),
        pltpu.VMEM((CHUNK, H), jnp.float32),
        pltpu.VMEM((CHUNK, H), jnp.float32),
        pltpu.VMEM((E_PER_W, H), jnp.float32),
        pltpu.SemaphoreType.DMA,
        pltpu.SemaphoreType.DMA,
        pltpu.SemaphoreType.DMA,
    ],
)(_sc_gather_body)


def _mlp_body(g1_ref, g2_ref, e1_ref, s1_ref, s2_ref, noise_ref,
              wa_ref, wb_ref, wc_ref, wd_ref, bp_ref,
              w1_ref, b1_ref, w2_ref, b2_ref, w3_ref, b3_ref,
              wo_ref, bo_ref,
              act_ref, pi_ref):
    f32 = jnp.float32
    a1 = jnp.dot(e1_ref[0], wa_ref[...], preferred_element_type=f32)
    a2 = jnp.dot(e1_ref[0], wb_ref[...], preferred_element_type=f32)
    h = (jnp.dot(s1_ref[...], a1, preferred_element_type=f32)
         + jnp.dot(s2_ref[...], a2, preferred_element_type=f32)
         + jnp.dot(g1_ref[0, 0], wc_ref[...], preferred_element_type=f32)
         + jnp.dot(g2_ref[0, 0], wd_ref[...], preferred_element_type=f32)
         + bp_ref[...])
    h = jnp.maximum(jnp.dot(h, w1_ref[...], preferred_element_type=f32) + b1_ref[...], 0.0)
    h = jnp.maximum(jnp.dot(h, w2_ref[...], preferred_element_type=f32) + b2_ref[...], 0.0)
    h = jnp.maximum(jnp.dot(h, w3_ref[...], preferred_element_type=f32) + b3_ref[...], 0.0)
    logits = jnp.dot(h, wo_ref[...], preferred_element_type=f32) + bo_ref[...]
    rowid = lax.broadcasted_iota(jnp.int32, (PP, 1), 0)
    logits = jnp.where(rowid < P, logits, f32(-1e30))
    z = logits + noise_ref[0]
    maxz = jnp.max(z)
    action = jnp.min(jnp.where(z >= maxz, rowid, jnp.int32(PP)))
    m = jnp.max(logits)
    lse = m + jnp.log(jnp.sum(jnp.exp(logits - m)))
    logit_a = jnp.sum(jnp.where(rowid == action, logits, 0.0))
    act_ref[0] = action[None, None]
    pi_ref[0] = (logit_a - lse)[None, None]


def _full(shape):
    nd = len(shape)
    return pl.BlockSpec(shape, lambda *a: (0,) * nd)


def kernel(x_edges, x_edges_values, x_nodes_coord, x_tour, x_best_tour,
           x_tour_directed, params):
    p = params
    f32 = jnp.float32
    cbn = np.float32(1.0 / np.sqrt(1.0 + 1e-5))
    xt = x_tour.astype(jnp.int32)
    xb = x_best_tour.astype(jnp.int32)

    vals4 = x_edges_values.reshape(B, V, V, 1)
    t4 = xt.reshape(B, V, V, 1)
    b4 = xb.reshape(B, V, V, 1)
    wev = p['W_evals'].reshape(1, HH)

    node_w_specs = [_full((H, H)), _full((1, H)),
                    _full((H, H)), _full((1, H)),
                    _full((H, H)), _full((1, H))]
    bvh_spec = pl.BlockSpec((1, V, H), lambda b: (b, 0, 0))

    par2 = pltpu.CompilerParams(dimension_semantics=("parallel", "parallel"))
    coords4 = x_nodes_coord.reshape(B, V, 1, 2)
    row_spec = pl.BlockSpec((1, RT, 1, H), lambda b, i: (b, i, 0, 0))
    w_spec = pl.BlockSpec((H, H), lambda b, i: (0, 0))
    h_spec = pl.BlockSpec((1, H), lambda b, i: (0, 0))
    xf_spec = pl.BlockSpec((1, V, H), lambda b, i: (b, 0, 0))
    e_spec = pl.BlockSpec((1, RT, V, H), lambda b, i: (b, i, 0, 0))
    hh_spec = pl.BlockSpec((1, HH), lambda b, i: (0, 0))
    emb_spec = pl.BlockSpec((3, HH), lambda b, i: (0, 0))

    e = None
    x4 = None
    for li, lp in enumerate(p['layers']):
        last = li == len(p['layers']) - 1
        ge = (lp['bn_e'][0] * cbn).reshape(1, H)
        be = lp['bn_e'][1].reshape(1, H)
        if last:
            e = pl.pallas_call(
                _edge_last_body,
                grid=(B, V // RT),
                in_specs=[e_spec, xf_spec, row_spec,
                          w_spec, h_spec, w_spec, h_spec, h_spec, h_spec],
                out_specs=e_spec,
                out_shape=jax.ShapeDtypeStruct((B, V, V, H), f32),
                compiler_params=par2,
            )(e, x4.reshape(B, V, H), x4,
              lp['Ve'][0], lp['Ve'][1].reshape(1, H),
              lp['Ue'][0], lp['Ue'][1].reshape(1, H), ge, be)
            break

        gn = (lp['bn_n'][0] * cbn).reshape(1, H)
        bn = lp['bn_n'][1].reshape(1, H)
        node_w_args = (lp['Ve'][0], lp['Ve'][1].reshape(1, H),
                       lp['Un'][0], lp['Un'][1].reshape(1, H),
                       lp['Vn'][0], lp['Vn'][1].reshape(1, H))
        node_w_sp = [w_spec, h_spec, w_spec, h_spec, w_spec, h_spec]
        tail_sp = node_w_sp + [w_spec, h_spec, h_spec, h_spec, h_spec, h_spec]
        tail_args = node_w_args + (lp['Ue'][0], lp['Ue'][1].reshape(1, H),
                                   ge, be, gn, bn)
        out_specs = [e_spec, row_spec]
        out_shape = [jax.ShapeDtypeStruct((B, V, V, H), f32),
                     jax.ShapeDtypeStruct((B, V, 1, H), f32)]
        if li == 0:
            e, x4 = pl.pallas_call(
                _edge1_body,
                grid=(B, V // RT),
                in_specs=[
                    pl.BlockSpec((1, RT, V, 1), lambda b, i: (b, i, 0, 0)),
                    pl.BlockSpec((1, RT, V, 1), lambda b, i: (b, i, 0, 0)),
                    pl.BlockSpec((1, RT, V, 1), lambda b, i: (b, i, 0, 0)),
                    hh_spec, emb_spec, emb_spec,
                    pl.BlockSpec((1, V, 2), lambda b, i: (b, 0, 0)),
                    pl.BlockSpec((1, RT, 1, 2), lambda b, i: (b, i, 0, 0)),
                    pl.BlockSpec((2, H), lambda b, i: (0, 0)),
                ] + tail_sp,
                out_specs=out_specs,
                out_shape=out_shape,
                compiler_params=par2,
            )(vals4, t4, b4, wev, p['emb0'], p['emb1'],
              x_nodes_coord, coords4, p['W_nodes'], *tail_args)
        else:
            e, x4 = pl.pallas_call(
                _edge_body,
                grid=(B, V // RT),
                in_specs=[e_spec, xf_spec, row_spec] + tail_sp,
                out_specs=out_specs,
                out_shape=out_shape,
                compiler_params=par2,
            )(e, x4.reshape(B, V, H), x4, *tail_args)

    first = jnp.argmax(xt, axis=2).astype(jnp.int32)
    last = (V - 1) - jnp.argmax(xt[:, :, ::-1], axis=2).astype(jnp.int32)
    ii = jnp.arange(V, dtype=jnp.int32)[None, :]
    cnt = (first > ii).astype(jnp.int32) + (last > ii).astype(jnp.int32)
    start = jnp.cumsum(cnt, axis=1) - cnt
    kk = jnp.arange(V, dtype=jnp.int32)
    i_e = jnp.sum((start[:, :, None] <= kk[None, None, :]).astype(jnp.int32),
                  axis=1) - 1
    f_i = jnp.take_along_axis(first, i_e, axis=1)
    l_i = jnp.take_along_axis(last, i_e, axis=1)
    s_i = jnp.take_along_axis(start, i_e, axis=1)
    firstj = jnp.where(f_i > i_e, f_i, l_i)
    j_e = jnp.where(kk[None, :] == s_i, firstj, l_i)

    d = jnp.take_along_axis(x_tour_directed.reshape(B, V * V),
                            i_e * V + j_e, axis=1)
    U = jnp.where(d, i_e, j_e)
    Vv = jnp.where(d, j_e, i_e)

    boff = (jnp.arange(B, dtype=jnp.int32) * (V * V))[:, None]
    Uk1, Uk2 = U[:, RS_PAD], U[:, CS_PAD]
    Vk1, Vk2 = Vv[:, RS_PAD], Vv[:, CS_PAD]
    idx_g = jnp.stack([
        boff + Uk1 * V + Uk2,
        boff + Vk1 * V + Vk2,
    ]).reshape(NW, NCHUNK, CHUNK)
    idx_e = jnp.pad((boff + U * V + Vv).reshape(B * V),
                    (0, E_ROWS - B * V)).reshape(NW, E_PER_W)

    table = e.reshape(B * V * V, H)
    rows_g, rows_e = _sc_gather(table, idx_g, idx_e)
    quad = rows_g.reshape(2, B, PP, H)
    e1 = rows_e[:B * V].reshape(B, V, H)

    noise = jax.random.gumbel(jax.random.key(42), (B, P), f32)
    noise = jnp.pad(noise, ((0, 0), (0, PP - P))).reshape(B, PP, 1)
    Wp, bp = p['pre_act']
    w1, b1 = p['act_hidden'][0]
    w2, b2 = p['act_hidden'][1]
    w3, b3 = p['act_hidden'][2]
    wo, bo = p['act_out']
    tab_spec = lambda t: pl.BlockSpec((1, 1, PP, H), lambda b, _t=t: (_t, b, 0, 0))
    act2, pi2 = pl.pallas_call(
        _mlp_body,
        grid=(B,),
        in_specs=[
            tab_spec(0), tab_spec(1),
            pl.BlockSpec((1, V, H), lambda b: (b, 0, 0)),
            _full((PP, V)), _full((PP, V)),
            pl.BlockSpec((1, PP, 1), lambda b: (b, 0, 0)),
            _full((H, H)), _full((H, H)), _full((H, H)), _full((H, H)),
            _full((1, H)),
            _full((H, H)), _full((1, H)),
            _full((H, H)), _full((1, H)),
            _full((H, H)), _full((1, H)),
            _full((H, 1)), _full((1, 1)),
        ],
        out_specs=[pl.BlockSpec((1, 1, 1), lambda b: (b, 0, 0)),
                   pl.BlockSpec((1, 1, 1), lambda b: (b, 0, 0))],
        out_shape=[jax.ShapeDtypeStruct((B, 1, 1), jnp.int32),
                   jax.ShapeDtypeStruct((B, 1, 1), f32)],
    )(quad, quad, e1, jnp.asarray(S1_ONEHOT), jnp.asarray(S2_ONEHOT), noise,
      Wp[0:H], Wp[H:2 * H], Wp[2 * H:3 * H], Wp[3 * H:4 * H], bp.reshape(1, H),
      w1, b1.reshape(1, H), w2, b2.reshape(1, H), w3, b3.reshape(1, H),
      wo, bo.reshape(1, 1))

    actions = act2[:, 0, 0]
    pi = pi2[:, 0, 0]

    k1 = jnp.asarray(RS_PAD)[actions]
    k2 = jnp.asarray(CS_PAD)[actions]
    barange = jnp.arange(B, dtype=jnp.int32)

    def edge_row(kidx):
        return jnp.stack([
            barange,
            jnp.take_along_axis(i_e, kidx[:, None], axis=1)[:, 0],
            jnp.take_along_axis(j_e, kidx[:, None], axis=1)[:, 0],
        ], axis=1)

    edges = jnp.stack([edge_row(k1), edge_row(k2)], axis=1)
    return edges, pi, actions
